# Initial kernel scaffold; baseline (speedup 1.0000x reference)
import jax
import jax.numpy as jnp
from jax.experimental import pallas as pl


def kernel(x, edge_index, W_enc, b_enc, Wg1, bg1, Wg2, bg2, Wg3, bg3, Ws1, bs1, Ws2, bs2, Wv1, bv1, Wv2, bv2):
    raise NotImplementedError("write your pallas kernel here")


# speedup vs baseline: 14.2906x; 14.2906x over previous
import functools

import jax
import jax.numpy as jnp
from jax import lax
from jax.experimental import pallas as pl
from jax.experimental.pallas import tpu as pltpu
from jax.experimental.pallas import tpu_sc as plsc

N = 50000
E = 800000
D_IN = 128

NCORES = 2
NSUB = 16
NW = NCORES * NSUB
CHUNK = 128
E_PAD = 802816
EPT = E_PAD // NW
NCH = EPT // CHUNK
NP = 50048
STRIPE = NP // NSUB

BLK = 512
GRID = (NP + BLK - 1) // BLK

SFULL = STRIPE // CHUNK
STAIL = STRIPE - SFULL * CHUNK


def _zero_stripe(rows, acc, r0):
    def zb(t, carry):
        off = pl.multiple_of(r0 + t * CHUNK, 8)
        pltpu.sync_copy(rows, acc.at[pl.ds(off, CHUNK)])
        return carry
    lax.fori_loop(0, SFULL, zb, 0)
    pltpu.sync_copy(rows.at[pl.ds(0, STAIL)],
                    acc.at[pl.ds(r0 + SFULL * CHUNK, STAIL)])


def _write_stripe(rows, acc, out_hbm, r0, obase):
    def wb(t, carry):
        off = pl.multiple_of(r0 + t * CHUNK, 8)
        pltpu.sync_copy(acc.at[pl.ds(off, CHUNK)], rows)
        pltpu.sync_copy(rows, out_hbm.at[pl.ds(obase + off, CHUNK)])
        return carry
    lax.fori_loop(0, SFULL, wb, 0)
    toff = r0 + SFULL * CHUNK
    pltpu.sync_copy(acc.at[pl.ds(toff, STAIL)], rows.at[pl.ds(0, STAIL)])
    pltpu.sync_copy(rows.at[pl.ds(0, STAIL)], out_hbm.at[pl.ds(obase + toff, STAIL)])


def _mesh():
    return plsc.VectorSubcoreMesh(
        core_axis_name="c", subcore_axis_name="s",
        num_cores=NCORES, num_subcores=NSUB)


def _deg_partials(dstp, zeros1):
    @functools.partial(
        pl.kernel,
        out_type=jax.ShapeDtypeStruct((2 * NP,), jnp.float32),
        mesh=_mesh(),
        compiler_params=pltpu.CompilerParams(use_tc_tiling_on_sc=False, needs_layout_passes=False),
        scratch_types=[
            pltpu.VMEM((CHUNK,), jnp.int32),
            pltpu.VMEM((CHUNK,), jnp.float32),
            pltpu.VMEM_SHARED((NP,), jnp.float32),
        ],
    )
    def k(dst_hbm, zero_hbm, out_hbm, didx, buf, acc):
        c = lax.axis_index("c")
        s = lax.axis_index("s")
        wid = c * NSUB + s
        r0 = s * STRIPE
        pltpu.sync_copy(zero_hbm, buf)
        _zero_stripe(buf, acc, r0)
        plsc.subcore_barrier()
        for i in range(CHUNK // 16):
            buf[pl.ds(16 * i, 16)] = jnp.full((16,), 1.0, jnp.float32)
        ebase = wid * EPT

        def body(i, carry):
            eb = pl.multiple_of(ebase + i * CHUNK, CHUNK)
            pltpu.sync_copy(dst_hbm.at[pl.ds(eb, CHUNK)], didx)
            pltpu.sync_copy(buf, acc.at[didx], add=True)
            return carry

        lax.fori_loop(0, NCH, body, 0)
        plsc.subcore_barrier()
        _write_stripe(buf, acc, out_hbm, r0, c * NP)

    return k(dstp, zeros1)


def _seg_sum_partials(mp, srcp, dstp, zeros2, F):
    @functools.partial(
        pl.kernel,
        out_type=jax.ShapeDtypeStruct((2 * NP, F), jnp.float32),
        mesh=_mesh(),
        compiler_params=pltpu.CompilerParams(use_tc_tiling_on_sc=False, needs_layout_passes=False),
        scratch_types=[
            pltpu.VMEM((CHUNK,), jnp.int32),
            pltpu.VMEM((CHUNK,), jnp.int32),
            pltpu.VMEM((CHUNK, F), jnp.float32),
            pltpu.VMEM_SHARED((NP, F), jnp.float32),
            pltpu.SemaphoreType.DMA,
        ],
    )
    def k(mp_hbm, src_hbm, dst_hbm, zero_hbm, out_hbm, sidx, didx, rows, acc, sem):
        c = lax.axis_index("c")
        s = lax.axis_index("s")
        wid = c * NSUB + s
        r0 = s * STRIPE
        pltpu.sync_copy(zero_hbm, rows)
        _zero_stripe(rows, acc, r0)
        plsc.subcore_barrier()
        ebase = wid * EPT

        def body(i, carry):
            eb = pl.multiple_of(ebase + i * CHUNK, CHUNK)
            pltpu.sync_copy(src_hbm.at[pl.ds(eb, CHUNK)], sidx)
            pltpu.sync_copy(dst_hbm.at[pl.ds(eb, CHUNK)], didx)
            pltpu.async_copy(mp_hbm.at[sidx], rows, sem).wait()
            pltpu.sync_copy(rows, acc.at[didx], add=True)
            return carry

        lax.fori_loop(0, NCH, body, 0)
        plsc.subcore_barrier()
        _write_stripe(rows, acc, out_hbm, r0, c * NP)

    return k(mp, srcp, dstp, zeros2)


def _edge_head(A, B, srcp, dstp, w2s, b2s):
    @functools.partial(
        pl.kernel,
        out_type=jax.ShapeDtypeStruct((E_PAD,), jnp.float32),
        mesh=_mesh(),
        compiler_params=pltpu.CompilerParams(use_tc_tiling_on_sc=False, needs_layout_passes=False),
        scratch_types=[
            pltpu.VMEM((CHUNK,), jnp.int32),
            pltpu.VMEM((CHUNK,), jnp.int32),
            pltpu.VMEM((CHUNK, 8), jnp.float32),
            pltpu.VMEM((CHUNK, 8), jnp.float32),
            pltpu.VMEM((8, 16), jnp.float32),
            pltpu.VMEM((16,), jnp.float32),
            pltpu.VMEM((CHUNK,), jnp.float32),
            pltpu.SemaphoreType.DMA,
            pltpu.SemaphoreType.DMA,
        ],
    )
    def k(a_hbm, b_hbm, src_hbm, dst_hbm, w_hbm, b2_hbm, out_hbm,
          sidx, didx, ra, rb, wv, b2v, ob, sem_a, sem_b):
        c = lax.axis_index("c")
        s = lax.axis_index("s")
        wid = c * NSUB + s
        pltpu.sync_copy(w_hbm, wv)
        pltpu.sync_copy(b2_hbm, b2v)
        wsp = [wv[j, :] for j in range(8)]
        b2 = b2v[...]
        iota = lax.iota(jnp.int32, 16)
        ebase = wid * EPT

        def body(i, carry):
            eb = pl.multiple_of(ebase + i * CHUNK, CHUNK)
            pltpu.sync_copy(src_hbm.at[pl.ds(eb, CHUNK)], sidx)
            pltpu.sync_copy(dst_hbm.at[pl.ds(eb, CHUNK)], didx)
            cp_a = pltpu.async_copy(a_hbm.at[sidx], ra, sem_a)
            cp_b = pltpu.async_copy(b_hbm.at[didx], rb, sem_b)
            cp_a.wait()
            cp_b.wait()
            for g in range(CHUNK // 16):
                ridx = iota + (g * 16)
                acc = jnp.zeros((16,), jnp.float32)
                for j in range(8):
                    cj = jnp.full((16,), j, jnp.int32)
                    av = plsc.load_gather(ra, [ridx, cj])
                    bv = plsc.load_gather(rb, [ridx, cj])
                    acc = acc + jnp.maximum(av + bv, 0.0) * wsp[j]
                t = acc + b2
                ob[pl.ds(g * 16, 16)] = 1.0 / (1.0 + jnp.exp(-t))
            pltpu.sync_copy(ob, out_hbm.at[pl.ds(eb, CHUNK)])
            return carry

        lax.fori_loop(0, NCH, body, 0)

    return k(A, B, srcp, dstp, w2s, b2s)


def _tc_encode(x, degp2, W_enc, b_enc, Wg1):
    def body(xb, degb, we, be, wg, mp_o, dis_o):
        deg = degb[0] + degb[1] + 1.0
        dis = lax.rsqrt(deg)
        h0 = jnp.maximum(jnp.dot(xb[...], we[...],
                                 preferred_element_type=jnp.float32) + be[...], 0.0)
        mp_o[...] = jnp.dot(h0, wg[...], preferred_element_type=jnp.float32) * dis[:, None]
        dis_o[...] = dis

    return pl.pallas_call(
        body,
        grid=(GRID,),
        in_specs=[
            pl.BlockSpec((BLK, D_IN), lambda i: (i, 0)),
            pl.BlockSpec((2, BLK), lambda i: (0, i)),
            pl.BlockSpec((D_IN, 64), lambda i: (0, 0)),
            pl.BlockSpec((64,), lambda i: (0,)),
            pl.BlockSpec((64, 32), lambda i: (0, 0)),
        ],
        out_specs=[
            pl.BlockSpec((BLK, 32), lambda i: (i, 0)),
            pl.BlockSpec((BLK,), lambda i: (i,)),
        ],
        out_shape=[
            jax.ShapeDtypeStruct((NP, 32), jnp.float32),
            jax.ShapeDtypeStruct((NP,), jnp.float32),
        ],
    )(x, degp2, W_enc, b_enc, Wg1)


def _tc_mid(accp, mp, dis, bg, Wn, Fi, Fo):
    def body(ab, mb, db, bgb, wb, o):
        d = db[...]
        h = jnp.maximum(d[:, None] * (ab[0] + ab[1] + mb[...]) + bgb[...], 0.0)
        o[...] = jnp.dot(h, wb[...], preferred_element_type=jnp.float32) * d[:, None]

    return pl.pallas_call(
        body,
        grid=(GRID,),
        in_specs=[
            pl.BlockSpec((2, BLK, Fi), lambda i: (0, i, 0)),
            pl.BlockSpec((BLK, Fi), lambda i: (i, 0)),
            pl.BlockSpec((BLK,), lambda i: (i,)),
            pl.BlockSpec((Fi,), lambda i: (0,)),
            pl.BlockSpec((Fi, Fo), lambda i: (0, 0)),
        ],
        out_specs=pl.BlockSpec((BLK, Fo), lambda i: (i, 0)),
        out_shape=jax.ShapeDtypeStruct((NP, Fo), jnp.float32),
    )(accp, mp, dis, bg, Wn)


def _tc_heads(accp, mp, dis, bg3, Ws1, bs1, Wv1, bv1, Wv2, bv2):
    def body(ab, mb, db, bgb, ws1, bs1b, wv1, bv1b, wv2, bv2b, a_o, b_o, vsq_o):
        d = db[...]
        h3 = d[:, None] * (ab[0] + ab[1] + mb[...]) + bgb[...]
        ws1v = ws1[...]
        a_o[...] = jnp.dot(h3, ws1v[:8], preferred_element_type=jnp.float32) + bs1b[...]
        b_o[...] = jnp.dot(h3, ws1v[8:], preferred_element_type=jnp.float32)
        z = jnp.maximum(jnp.dot(h3, wv1[...], preferred_element_type=jnp.float32)
                        + bv1b[...], 0.0)
        t = jnp.dot(z, wv2[...], preferred_element_type=jnp.float32) + bv2b[...]
        vs = 1.0 / (1.0 + jnp.exp(-t[:, 0]))
        vp = 0.9 + 0.2 * vs
        vsq_o[...] = vp * vp

    return pl.pallas_call(
        body,
        grid=(GRID,),
        in_specs=[
            pl.BlockSpec((2, BLK, 8), lambda i: (0, i, 0)),
            pl.BlockSpec((BLK, 8), lambda i: (i, 0)),
            pl.BlockSpec((BLK,), lambda i: (i,)),
            pl.BlockSpec((8,), lambda i: (0,)),
            pl.BlockSpec((16, 8), lambda i: (0, 0)),
            pl.BlockSpec((8,), lambda i: (0,)),
            pl.BlockSpec((8, 4), lambda i: (0, 0)),
            pl.BlockSpec((4,), lambda i: (0,)),
            pl.BlockSpec((4, 1), lambda i: (0, 0)),
            pl.BlockSpec((1,), lambda i: (0,)),
        ],
        out_specs=[
            pl.BlockSpec((BLK, 8), lambda i: (i, 0)),
            pl.BlockSpec((BLK, 8), lambda i: (i, 0)),
            pl.BlockSpec((BLK,), lambda i: (i,)),
        ],
        out_shape=[
            jax.ShapeDtypeStruct((NP, 8), jnp.float32),
            jax.ShapeDtypeStruct((NP, 8), jnp.float32),
            jax.ShapeDtypeStruct((NP,), jnp.float32),
        ],
    )(accp, mp, dis, bg3, Ws1, bs1, Wv1, bv1, Wv2, bv2)


def kernel(x, edge_index, W_enc, b_enc, Wg1, bg1, Wg2, bg2, Wg3, bg3,
           Ws1, bs1, Ws2, bs2, Wv1, bv1, Wv2, bv2):
    src = edge_index[0].astype(jnp.int32)
    dst = edge_index[1].astype(jnp.int32)
    pad = jnp.full((E_PAD - E,), N, jnp.int32)
    srcp = jnp.concatenate([src, pad])
    dstp = jnp.concatenate([dst, pad])

    degp = _deg_partials(dstp, jnp.zeros((CHUNK,), jnp.float32)).reshape(2, NP)
    mp1, dis = _tc_encode(x, degp, W_enc, b_enc, Wg1)

    acc1 = _seg_sum_partials(mp1, srcp, dstp,
                             jnp.zeros((CHUNK, 32), jnp.float32), 32).reshape(2, NP, 32)
    mp2 = _tc_mid(acc1, mp1, dis, bg1, Wg2, 32, 16)

    acc2 = _seg_sum_partials(mp2, srcp, dstp,
                             jnp.zeros((CHUNK, 16), jnp.float32), 16).reshape(2, NP, 16)
    mp3 = _tc_mid(acc2, mp2, dis, bg2, Wg3, 16, 8)

    acc3 = _seg_sum_partials(mp3, srcp, dstp,
                             jnp.zeros((CHUNK, 8), jnp.float32), 8).reshape(2, NP, 8)
    A, B, vsq = _tc_heads(acc3, mp3, dis, bg3, Ws1, bs1, Wv1, bv1, Wv2, bv2)

    w2s = jnp.tile(Ws2.reshape(8, 1), (1, 16))
    b2s = jnp.full((16,), 1.0, jnp.float32) * bs2[0]
    swp = _edge_head(A, B, srcp, dstp, w2s, b2s)

    return swp[:E], vsq[:N]


# speedup vs baseline: 30.4307x; 2.1294x over previous
import functools

import jax
import jax.numpy as jnp
from jax import lax
from jax.experimental import pallas as pl
from jax.experimental.pallas import tpu as pltpu
from jax.experimental.pallas import tpu_sc as plsc

N = 50000
E = 800000
D_IN = 128

NCORES = 2
NSUB = 16
NW = NCORES * NSUB
CHUNK = 128
E_PAD = 802816
EPT = E_PAD // NW
NCH = EPT // CHUNK
K = 28
NP = 50048
STRIPE = NP // NSUB

BLK = 512
GRID = (NP + BLK - 1) // BLK

SFULL = STRIPE // CHUNK
STAIL = STRIPE - SFULL * CHUNK
SCH = SFULL + 1


def _mesh():
    return plsc.VectorSubcoreMesh(
        core_axis_name="c", subcore_axis_name="s",
        num_cores=NCORES, num_subcores=NSUB)


def _zero_stripe(zrow, acc, r0, sem):
    cps = []
    for t in range(SFULL):
        off = pl.multiple_of(r0 + t * CHUNK, 8)
        cps.append(pltpu.async_copy(zrow, acc.at[pl.ds(off, CHUNK)], sem))
    cps.append(pltpu.async_copy(
        zrow.at[pl.ds(0, STAIL)],
        acc.at[pl.ds(r0 + SFULL * CHUNK, STAIL)], sem))
    for cp in cps:
        cp.wait()


def _write_stripe(buf, nslots, acc, out_hbm, r0, obase, sem_l, sem_w):
    chunks = [(t, CHUNK) for t in range(SFULL)] + [(SFULL, STAIL)]
    for w0 in range(0, SCH, nslots):
        wave = chunks[w0:w0 + nslots]
        cps = []
        for slot, (t, ln) in enumerate(wave):
            off = pl.multiple_of(r0 + t * CHUNK, 8)
            cps.append(pltpu.async_copy(acc.at[pl.ds(off, ln)],
                                        buf.at[slot, pl.ds(0, ln)], sem_l))
        for cp in cps:
            cp.wait()
        cps = []
        for slot, (t, ln) in enumerate(wave):
            off = pl.multiple_of(r0 + t * CHUNK, 8)
            cps.append(pltpu.async_copy(buf.at[slot, pl.ds(0, ln)],
                                        out_hbm.at[pl.ds(obase + off, ln)], sem_w))
        for cp in cps:
            cp.wait()


def _deg_partials(dst2p, zrow):
    @functools.partial(
        pl.kernel,
        out_type=jax.ShapeDtypeStruct((2 * NP,), jnp.float32),
        mesh=_mesh(),
        compiler_params=pltpu.CompilerParams(
            use_tc_tiling_on_sc=False, needs_layout_passes=False),
        scratch_types=[
            pltpu.VMEM((K, CHUNK), jnp.int32),
            pltpu.VMEM((CHUNK,), jnp.float32),
            pltpu.VMEM((SCH, CHUNK), jnp.float32),
            pltpu.VMEM_SHARED((NP,), jnp.float32),
            pltpu.SemaphoreType.DMA,
            pltpu.SemaphoreType.DMA,
        ],
    )
    def k(dst_hbm, zero_hbm, out_hbm, gdidx, ones, buf, acc, sem_i, sem_s):
        c = lax.axis_index("c")
        s = lax.axis_index("s")
        wid = c * NSUB + s
        r0 = pl.multiple_of(s * STRIPE, 8)
        pltpu.sync_copy(zero_hbm, ones)
        _zero_stripe(ones, acc, r0, sem_i)
        plsc.subcore_barrier()
        for i in range(CHUNK // 16):
            ones[pl.ds(16 * i, 16)] = jnp.full((16,), 1.0, jnp.float32)
        rbase = wid * NCH

        def body(g, carry):
            rb = rbase + g * K
            pltpu.async_copy(dst_hbm.at[pl.ds(rb, K)], gdidx, sem_i).wait()
            cps = [pltpu.async_copy(ones, acc.at[gdidx.at[b]], sem_s, add=True)
                   for b in range(K)]
            for cp in cps:
                cp.wait()
            return carry

        lax.fori_loop(0, NCH // K, body, 0)
        plsc.subcore_barrier()
        _write_stripe(buf, SCH, acc, out_hbm, r0, c * NP, sem_i, sem_s)

    return k(dst2p, zrow)


def _seg_sum_partials(mp, srcp, dst2p, zrow, F, kc):
    @functools.partial(
        pl.kernel,
        out_type=jax.ShapeDtypeStruct((2 * NP, F), jnp.float32),
        mesh=_mesh(),
        compiler_params=pltpu.CompilerParams(
            use_tc_tiling_on_sc=False, needs_layout_passes=False),
        scratch_types=[
            pltpu.VMEM((kc * CHUNK,), jnp.int32),
            pltpu.VMEM((kc, CHUNK), jnp.int32),
            pltpu.VMEM((kc, CHUNK, F), jnp.float32),
            pltpu.VMEM_SHARED((NP, F), jnp.float32),
            pltpu.SemaphoreType.DMA,
            pltpu.SemaphoreType.DMA,
            pltpu.SemaphoreType.DMA,
        ],
    )
    def k(mp_hbm, src_hbm, dst_hbm, zero_hbm, out_hbm,
          gsidx, gdidx, rows, acc, sem_i, sem_g, sem_s):
        c = lax.axis_index("c")
        s = lax.axis_index("s")
        wid = c * NSUB + s
        r0 = pl.multiple_of(s * STRIPE, 8)
        pltpu.sync_copy(zero_hbm, rows.at[0])
        _zero_stripe(rows.at[0], acc, r0, sem_i)
        plsc.subcore_barrier()
        ebase = wid * EPT
        rbase = wid * NCH

        def body(g, carry):
            eb = pl.multiple_of(ebase + g * (kc * CHUNK), 8)
            rb = rbase + g * kc
            cpi = pltpu.async_copy(src_hbm.at[pl.ds(eb, kc * CHUNK)], gsidx, sem_i)
            cpd = pltpu.async_copy(dst_hbm.at[pl.ds(rb, kc)], gdidx, sem_i)
            cpi.wait()
            cpd.wait()
            cps = [pltpu.async_copy(
                       mp_hbm.at[gsidx.at[pl.ds(b * CHUNK, CHUNK)]],
                       rows.at[b], sem_g)
                   for b in range(kc)]
            for cp in cps:
                cp.wait()
            cps = [pltpu.async_copy(rows.at[b], acc.at[gdidx.at[b]], sem_s, add=True)
                   for b in range(kc)]
            for cp in cps:
                cp.wait()
            return carry

        lax.fori_loop(0, NCH // kc, body, 0)
        plsc.subcore_barrier()
        _write_stripe(rows, kc, acc, out_hbm, r0, c * NP, sem_g, sem_s)

    return k(mp, srcp, dst2p, zrow)


def _edge_head(A, B, srcp, dstp, w2s, b2s):
    @functools.partial(
        pl.kernel,
        out_type=jax.ShapeDtypeStruct((E_PAD // CHUNK, CHUNK), jnp.float32),
        mesh=_mesh(),
        compiler_params=pltpu.CompilerParams(
            use_tc_tiling_on_sc=False, needs_layout_passes=False),
        scratch_types=[
            pltpu.VMEM((K * CHUNK,), jnp.int32),
            pltpu.VMEM((K * CHUNK,), jnp.int32),
            pltpu.VMEM((K, CHUNK, 8), jnp.float32),
            pltpu.VMEM((K, CHUNK, 8), jnp.float32),
            pltpu.VMEM((8, 16), jnp.float32),
            pltpu.VMEM((16,), jnp.float32),
            pltpu.VMEM((K, CHUNK), jnp.float32),
            pltpu.SemaphoreType.DMA,
            pltpu.SemaphoreType.DMA,
            pltpu.SemaphoreType.DMA,
        ],
    )
    def k(a_hbm, b_hbm, src_hbm, dst_hbm, w_hbm, b2_hbm, out_hbm,
          gsidx, gdidx, ra, rb, wv, b2v, ob, sem_i, sem_g, sem_w):
        c = lax.axis_index("c")
        s = lax.axis_index("s")
        wid = c * NSUB + s
        pltpu.sync_copy(w_hbm, wv)
        pltpu.sync_copy(b2_hbm, b2v)
        wsp = [wv[j, :] for j in range(8)]
        b2 = b2v[...]
        iota = lax.iota(jnp.int32, 16)
        ebase = wid * EPT
        rbase = wid * NCH

        def body(g, carry):
            eb = pl.multiple_of(ebase + g * (K * CHUNK), 8)
            rbrow = rbase + g * K
            cpi = pltpu.async_copy(src_hbm.at[pl.ds(eb, K * CHUNK)], gsidx, sem_i)
            cpd = pltpu.async_copy(dst_hbm.at[pl.ds(eb, K * CHUNK)], gdidx, sem_i)
            cpi.wait()
            cpd.wait()
            cps = []
            for b in range(K):
                sl = pl.ds(b * CHUNK, CHUNK)
                cps.append(pltpu.async_copy(a_hbm.at[gsidx.at[sl]], ra.at[b], sem_g))
                cps.append(pltpu.async_copy(b_hbm.at[gdidx.at[sl]], rb.at[b], sem_g))
            for cp in cps:
                cp.wait()

            def chunk(b, carry2):
                bb = jnp.full((16,), b, jnp.int32)
                for gg in range(CHUNK // 16):
                    ridx = iota + (gg * 16)
                    acc = jnp.zeros((16,), jnp.float32)
                    for j in range(8):
                        cj = jnp.full((16,), j, jnp.int32)
                        av = plsc.load_gather(ra, [bb, ridx, cj])
                        bv = plsc.load_gather(rb, [bb, ridx, cj])
                        acc = acc + jnp.maximum(av + bv, 0.0) * wsp[j]
                    t = acc + b2
                    ob[b, pl.ds(gg * 16, 16)] = 1.0 / (1.0 + jnp.exp(-t))
                return carry2

            lax.fori_loop(0, K, chunk, 0)
            pltpu.async_copy(ob, out_hbm.at[pl.ds(rbrow, K)], sem_w).wait()
            return carry

        lax.fori_loop(0, NCH // K, body, 0)

    return k(A, B, srcp, dstp, w2s, b2s)


def _tc_encode(x, degp2, W_enc, b_enc, Wg1):
    def body(xb, degb, we, be, wg, mp_o, dis_o):
        deg = degb[0] + degb[1] + 1.0
        dis = lax.rsqrt(deg)
        h0 = jnp.maximum(jnp.dot(xb[...], we[...],
                                 preferred_element_type=jnp.float32) + be[...], 0.0)
        mp_o[...] = jnp.dot(h0, wg[...], preferred_element_type=jnp.float32) * dis[:, None]
        dis_o[...] = dis

    return pl.pallas_call(
        body,
        grid=(GRID,),
        in_specs=[
            pl.BlockSpec((BLK, D_IN), lambda i: (i, 0)),
            pl.BlockSpec((2, BLK), lambda i: (0, i)),
            pl.BlockSpec((D_IN, 64), lambda i: (0, 0)),
            pl.BlockSpec((64,), lambda i: (0,)),
            pl.BlockSpec((64, 32), lambda i: (0, 0)),
        ],
        out_specs=[
            pl.BlockSpec((BLK, 32), lambda i: (i, 0)),
            pl.BlockSpec((BLK,), lambda i: (i,)),
        ],
        out_shape=[
            jax.ShapeDtypeStruct((NP, 32), jnp.float32),
            jax.ShapeDtypeStruct((NP,), jnp.float32),
        ],
    )(x, degp2, W_enc, b_enc, Wg1)


def _tc_mid(accp, mp, dis, bg, Wn, Fi, Fo):
    def body(ab, mb, db, bgb, wb, o):
        d = db[...]
        h = jnp.maximum(d[:, None] * (ab[0] + ab[1] + mb[...]) + bgb[...], 0.0)
        o[...] = jnp.dot(h, wb[...], preferred_element_type=jnp.float32) * d[:, None]

    return pl.pallas_call(
        body,
        grid=(GRID,),
        in_specs=[
            pl.BlockSpec((2, BLK, Fi), lambda i: (0, i, 0)),
            pl.BlockSpec((BLK, Fi), lambda i: (i, 0)),
            pl.BlockSpec((BLK,), lambda i: (i,)),
            pl.BlockSpec((Fi,), lambda i: (0,)),
            pl.BlockSpec((Fi, Fo), lambda i: (0, 0)),
        ],
        out_specs=pl.BlockSpec((BLK, Fo), lambda i: (i, 0)),
        out_shape=jax.ShapeDtypeStruct((NP, Fo), jnp.float32),
    )(accp, mp, dis, bg, Wn)


def _tc_heads(accp, mp, dis, bg3, Ws1, bs1, Wv1, bv1, Wv2, bv2):
    def body(ab, mb, db, bgb, ws1, bs1b, wv1, bv1b, wv2, bv2b, a_o, b_o, vsq_o):
        d = db[...]
        h3 = d[:, None] * (ab[0] + ab[1] + mb[...]) + bgb[...]
        ws1v = ws1[...]
        a_o[...] = jnp.dot(h3, ws1v[:8], preferred_element_type=jnp.float32) + bs1b[...]
        b_o[...] = jnp.dot(h3, ws1v[8:], preferred_element_type=jnp.float32)
        z = jnp.maximum(jnp.dot(h3, wv1[...], preferred_element_type=jnp.float32)
                        + bv1b[...], 0.0)
        t = jnp.dot(z, wv2[...], preferred_element_type=jnp.float32) + bv2b[...]
        vs = 1.0 / (1.0 + jnp.exp(-t[:, 0]))
        vp = 0.9 + 0.2 * vs
        vsq_o[...] = vp * vp

    return pl.pallas_call(
        body,
        grid=(GRID,),
        in_specs=[
            pl.BlockSpec((2, BLK, 8), lambda i: (0, i, 0)),
            pl.BlockSpec((BLK, 8), lambda i: (i, 0)),
            pl.BlockSpec((BLK,), lambda i: (i,)),
            pl.BlockSpec((8,), lambda i: (0,)),
            pl.BlockSpec((16, 8), lambda i: (0, 0)),
            pl.BlockSpec((8,), lambda i: (0,)),
            pl.BlockSpec((8, 4), lambda i: (0, 0)),
            pl.BlockSpec((4,), lambda i: (0,)),
            pl.BlockSpec((4, 1), lambda i: (0, 0)),
            pl.BlockSpec((1,), lambda i: (0,)),
        ],
        out_specs=[
            pl.BlockSpec((BLK, 8), lambda i: (i, 0)),
            pl.BlockSpec((BLK, 8), lambda i: (i, 0)),
            pl.BlockSpec((BLK,), lambda i: (i,)),
        ],
        out_shape=[
            jax.ShapeDtypeStruct((NP, 8), jnp.float32),
            jax.ShapeDtypeStruct((NP, 8), jnp.float32),
            jax.ShapeDtypeStruct((NP,), jnp.float32),
        ],
    )(accp, mp, dis, bg3, Ws1, bs1, Wv1, bv1, Wv2, bv2)


def kernel(x, edge_index, W_enc, b_enc, Wg1, bg1, Wg2, bg2, Wg3, bg3,
           Ws1, bs1, Ws2, bs2, Wv1, bv1, Wv2, bv2):
    src = edge_index[0].astype(jnp.int32)
    dst = edge_index[1].astype(jnp.int32)
    pad = jnp.full((E_PAD - E,), N, jnp.int32)
    srcp = jnp.concatenate([src, pad])
    dstp = jnp.concatenate([dst, pad])
    dst2p = dstp.reshape(E_PAD // CHUNK, CHUNK)

    degp = _deg_partials(dst2p, jnp.zeros((CHUNK,), jnp.float32)).reshape(2, NP)
    mp1, dis = _tc_encode(x, degp, W_enc, b_enc, Wg1)

    acc1 = _seg_sum_partials(mp1, srcp, dst2p, jnp.zeros((CHUNK, 32), jnp.float32),
                             32, 7).reshape(2, NP, 32)
    mp2 = _tc_mid(acc1, mp1, dis, bg1, Wg2, 32, 16)

    acc2 = _seg_sum_partials(mp2, srcp, dst2p, jnp.zeros((CHUNK, 16), jnp.float32),
                             16, 28).reshape(2, NP, 16)
    mp3 = _tc_mid(acc2, mp2, dis, bg2, Wg3, 16, 8)

    acc3 = _seg_sum_partials(mp3, srcp, dst2p, jnp.zeros((CHUNK, 8), jnp.float32),
                             8, 28).reshape(2, NP, 8)
    A, B, vsq = _tc_heads(acc3, mp3, dis, bg3, Ws1, bs1, Wv1, bv1, Wv2, bv2)

    w2s = jnp.tile(Ws2.reshape(8, 1), (1, 16))
    b2s = jnp.full((16,), 1.0, jnp.float32) * bs2[0]
    swp = _edge_head(A, B, srcp, dstp, w2s, b2s)

    return swp.reshape(E_PAD)[:E], vsq[:N]


# speedup vs baseline: 40.4020x; 1.3277x over previous
import functools

import jax
import jax.numpy as jnp
from jax import lax
from jax.experimental import pallas as pl
from jax.experimental.pallas import tpu as pltpu
from jax.experimental.pallas import tpu_sc as plsc

N = 50000
E = 800000
D_IN = 128

NCORES = 2
NSUB = 16
NW = NCORES * NSUB
CHUNK = 128
EPT = E // NW
NFC = EPT // CHUNK
TAIL = EPT - NFC * CHUNK
K = 28
NP = 50176
STRIPE = NP // NSUB

BLK = 6272
GRID = NP // BLK

SFULL = STRIPE // CHUNK
STAIL = STRIPE - SFULL * CHUNK
SCH = SFULL + 1


def _mesh():
    return plsc.VectorSubcoreMesh(
        core_axis_name="c", subcore_axis_name="s",
        num_cores=NCORES, num_subcores=NSUB)


def _zero_stripe(zrow, acc, r0, sem):
    cps = []
    for t in range(SFULL):
        off = pl.multiple_of(r0 + t * CHUNK, 8)
        cps.append(pltpu.async_copy(zrow, acc.at[pl.ds(off, CHUNK)], sem))
    cps.append(pltpu.async_copy(
        zrow.at[pl.ds(0, STAIL)],
        acc.at[pl.ds(r0 + SFULL * CHUNK, STAIL)], sem))
    for cp in cps:
        cp.wait()


def _write_stripe(buf, nslots, acc, out_hbm, r0, obase, sem_l, sem_w):
    chunks = [(t, CHUNK) for t in range(SFULL)] + [(SFULL, STAIL)]
    for w0 in range(0, SCH, nslots):
        wave = chunks[w0:w0 + nslots]
        cps = []
        for slot, (t, ln) in enumerate(wave):
            off = pl.multiple_of(r0 + t * CHUNK, 8)
            cps.append(pltpu.async_copy(acc.at[pl.ds(off, ln)],
                                        buf.at[slot, pl.ds(0, ln)], sem_l))
        for cp in cps:
            cp.wait()
        cps = []
        for slot, (t, ln) in enumerate(wave):
            off = pl.multiple_of(r0 + t * CHUNK, 8)
            cps.append(pltpu.async_copy(buf.at[slot, pl.ds(0, ln)],
                                        out_hbm.at[pl.ds(obase + off, ln)], sem_w))
        for cp in cps:
            cp.wait()


def _deg_partials(ei, zrow):
    @functools.partial(
        pl.kernel,
        out_type=jax.ShapeDtypeStruct((2 * NP,), jnp.float32),
        mesh=_mesh(),
        compiler_params=pltpu.CompilerParams(
            use_tc_tiling_on_sc=False, needs_layout_passes=False),
        scratch_types=[
            pltpu.VMEM((K, CHUNK), jnp.int32),
            pltpu.VMEM((TAIL,), jnp.int32),
            pltpu.VMEM((CHUNK,), jnp.float32),
            pltpu.VMEM((SCH, CHUNK), jnp.float32),
            pltpu.VMEM_SHARED((NP,), jnp.float32),
            pltpu.SemaphoreType.DMA,
            pltpu.SemaphoreType.DMA,
        ],
    )
    def k(ei_hbm, zero_hbm, out_hbm, gdidx, tdidx, ones, buf, acc, sem_i, sem_s):
        c = lax.axis_index("c")
        s = lax.axis_index("s")
        wid = c * NSUB + s
        r0 = pl.multiple_of(s * STRIPE, 8)
        pltpu.sync_copy(zero_hbm, ones)
        _zero_stripe(ones, acc, r0, sem_i)
        plsc.subcore_barrier()
        for i in range(CHUNK // 16):
            ones[pl.ds(16 * i, 16)] = jnp.full((16,), 1.0, jnp.float32)
        ebase = wid * EPT

        def do_group(cbase, nch):
            eb = pl.multiple_of(ebase + cbase * CHUNK, 8)
            cps = [pltpu.async_copy(
                       ei_hbm.at[1, pl.ds(eb + b * CHUNK, CHUNK)],
                       gdidx.at[b], sem_i)
                   for b in range(nch)]
            for cp in cps:
                cp.wait()
            cps = [pltpu.async_copy(ones, acc.at[gdidx.at[b]], sem_s, add=True)
                   for b in range(nch)]
            for cp in cps:
                cp.wait()

        def body(g, carry):
            do_group(g * K, K)
            return carry

        gf = NFC // K
        lax.fori_loop(0, gf, body, 0)
        if NFC % K:
            do_group(gf * K, NFC % K)
        teb = pl.multiple_of(ebase + NFC * CHUNK, 8)
        pltpu.sync_copy(ei_hbm.at[1, pl.ds(teb, TAIL)], tdidx)
        pltpu.sync_copy(ones.at[pl.ds(0, TAIL)], acc.at[tdidx], add=True)
        plsc.subcore_barrier()
        _write_stripe(buf, SCH, acc, out_hbm, r0, c * NP, sem_i, sem_s)

    return k(ei, zrow)


def _seg_sum_partials(mp, ei, zrow, F, kc):
    @functools.partial(
        pl.kernel,
        out_type=jax.ShapeDtypeStruct((2 * NP, F), jnp.float32),
        mesh=_mesh(),
        compiler_params=pltpu.CompilerParams(
            use_tc_tiling_on_sc=False, needs_layout_passes=False),
        scratch_types=[
            pltpu.VMEM((kc * CHUNK,), jnp.int32),
            pltpu.VMEM((kc, CHUNK), jnp.int32),
            pltpu.VMEM((TAIL,), jnp.int32),
            pltpu.VMEM((TAIL,), jnp.int32),
            pltpu.VMEM((kc, CHUNK, F), jnp.float32),
            pltpu.VMEM_SHARED((NP, F), jnp.float32),
            pltpu.SemaphoreType.DMA,
            pltpu.SemaphoreType.DMA,
            pltpu.SemaphoreType.DMA,
        ],
    )
    def k(mp_hbm, ei_hbm, zero_hbm, out_hbm,
          gsidx, gdidx, tsidx, tdidx, rows, acc, sem_i, sem_g, sem_s):
        c = lax.axis_index("c")
        s = lax.axis_index("s")
        wid = c * NSUB + s
        r0 = pl.multiple_of(s * STRIPE, 8)
        pltpu.sync_copy(zero_hbm, rows.at[0])
        _zero_stripe(rows.at[0], acc, r0, sem_i)
        plsc.subcore_barrier()
        ebase = wid * EPT

        def do_group(cbase, nch):
            eb = pl.multiple_of(ebase + cbase * CHUNK, 8)
            cps = [pltpu.async_copy(ei_hbm.at[0, pl.ds(eb, nch * CHUNK)],
                                    gsidx.at[pl.ds(0, nch * CHUNK)], sem_i)]
            cps += [pltpu.async_copy(
                        ei_hbm.at[1, pl.ds(eb + b * CHUNK, CHUNK)],
                        gdidx.at[b], sem_i)
                    for b in range(nch)]
            for cp in cps:
                cp.wait()
            cps = [pltpu.async_copy(
                       mp_hbm.at[gsidx.at[pl.ds(b * CHUNK, CHUNK)]],
                       rows.at[b], sem_g)
                   for b in range(nch)]
            for cp in cps:
                cp.wait()
            cps = [pltpu.async_copy(rows.at[b], acc.at[gdidx.at[b]], sem_s, add=True)
                   for b in range(nch)]
            for cp in cps:
                cp.wait()

        def body(g, carry):
            do_group(g * kc, kc)
            return carry

        gf = NFC // kc
        lax.fori_loop(0, gf, body, 0)
        if NFC % kc:
            do_group(gf * kc, NFC % kc)
        teb = pl.multiple_of(ebase + NFC * CHUNK, 8)
        pltpu.sync_copy(ei_hbm.at[0, pl.ds(teb, TAIL)], tsidx)
        pltpu.sync_copy(ei_hbm.at[1, pl.ds(teb, TAIL)], tdidx)
        pltpu.async_copy(mp_hbm.at[tsidx], rows.at[0, pl.ds(0, TAIL)], sem_g).wait()
        pltpu.sync_copy(rows.at[0, pl.ds(0, TAIL)], acc.at[tdidx], add=True)
        plsc.subcore_barrier()
        _write_stripe(rows, kc, acc, out_hbm, r0, c * NP, sem_g, sem_s)

    return k(mp, ei, zrow)


def _edge_head(A, B, ei, w2s, b2s):
    @functools.partial(
        pl.kernel,
        out_type=jax.ShapeDtypeStruct((E,), jnp.float32),
        mesh=_mesh(),
        compiler_params=pltpu.CompilerParams(
            use_tc_tiling_on_sc=False, needs_layout_passes=False),
        scratch_types=[
            pltpu.VMEM((K * CHUNK,), jnp.int32),
            pltpu.VMEM((K * CHUNK,), jnp.int32),
            pltpu.VMEM((TAIL,), jnp.int32),
            pltpu.VMEM((TAIL,), jnp.int32),
            pltpu.VMEM((K, CHUNK, 8), jnp.float32),
            pltpu.VMEM((K, CHUNK, 8), jnp.float32),
            pltpu.VMEM((8, 16), jnp.float32),
            pltpu.VMEM((16,), jnp.float32),
            pltpu.VMEM((K * CHUNK,), jnp.float32),
            pltpu.SemaphoreType.DMA,
            pltpu.SemaphoreType.DMA,
            pltpu.SemaphoreType.DMA,
        ],
    )
    def k(a_hbm, b_hbm, ei_hbm, w_hbm, b2_hbm, out_hbm,
          gsidx, gdidx, tsidx, tdidx, ra, rb, wv, b2v, ob, sem_i, sem_g, sem_w):
        c = lax.axis_index("c")
        s = lax.axis_index("s")
        wid = c * NSUB + s
        pltpu.sync_copy(w_hbm, wv)
        pltpu.sync_copy(b2_hbm, b2v)
        wsp = [wv[j, :] for j in range(8)]
        b2 = b2v[...]
        iota = lax.iota(jnp.int32, 16)
        ebase = wid * EPT

        def compute_chunk(b, nvec):
            bb = jnp.full((16,), b, jnp.int32)
            for gg in range(nvec):
                ridx = iota + (gg * 16)
                acc = jnp.zeros((16,), jnp.float32)
                for j in range(8):
                    cj = jnp.full((16,), j, jnp.int32)
                    av = plsc.load_gather(ra, [bb, ridx, cj])
                    bv = plsc.load_gather(rb, [bb, ridx, cj])
                    acc = acc + jnp.maximum(av + bv, 0.0) * wsp[j]
                t = acc + b2
                ob[pl.ds(b * CHUNK + gg * 16, 16)] = 1.0 / (1.0 + jnp.exp(-t))

        def do_group(cbase, nch):
            eb = pl.multiple_of(ebase + cbase * CHUNK, 8)
            nw = nch * CHUNK
            cpi = pltpu.async_copy(ei_hbm.at[0, pl.ds(eb, nw)],
                                   gsidx.at[pl.ds(0, nw)], sem_i)
            cpd = pltpu.async_copy(ei_hbm.at[1, pl.ds(eb, nw)],
                                   gdidx.at[pl.ds(0, nw)], sem_i)
            cpi.wait()
            cpd.wait()
            cps = []
            for b in range(nch):
                sl = pl.ds(b * CHUNK, CHUNK)
                cps.append(pltpu.async_copy(a_hbm.at[gsidx.at[sl]], ra.at[b], sem_g))
                cps.append(pltpu.async_copy(b_hbm.at[gdidx.at[sl]], rb.at[b], sem_g))
            for cp in cps:
                cp.wait()

            def chunk(b, carry2):
                compute_chunk(b, CHUNK // 16)
                return carry2

            lax.fori_loop(0, nch, chunk, 0)
            pltpu.async_copy(ob.at[pl.ds(0, nw)],
                             out_hbm.at[pl.ds(eb, nw)], sem_w).wait()

        def body(g, carry):
            do_group(g * K, K)
            return carry

        gf = NFC // K
        lax.fori_loop(0, gf, body, 0)
        if NFC % K:
            do_group(gf * K, NFC % K)
        teb = pl.multiple_of(ebase + NFC * CHUNK, 8)
        pltpu.sync_copy(ei_hbm.at[0, pl.ds(teb, TAIL)], tsidx)
        pltpu.sync_copy(ei_hbm.at[1, pl.ds(teb, TAIL)], tdidx)
        cpa = pltpu.async_copy(a_hbm.at[tsidx], ra.at[0, pl.ds(0, TAIL)], sem_g)
        cpb = pltpu.async_copy(b_hbm.at[tdidx], rb.at[0, pl.ds(0, TAIL)], sem_g)
        cpa.wait()
        cpb.wait()
        compute_chunk(0, (TAIL + 15) // 16)
        pltpu.sync_copy(ob.at[pl.ds(0, TAIL)], out_hbm.at[pl.ds(teb, TAIL)])

    return k(A, B, ei, w2s, b2s)


def _tc_encode(x, degp2, W_enc, b_enc, Wg1):
    def body(xb, degb, we, be, wg, mp_o, dis_o):
        deg = degb[0] + degb[1] + 1.0
        dis = lax.rsqrt(deg)
        h0 = jnp.maximum(jnp.dot(xb[...], we[...],
                                 preferred_element_type=jnp.float32) + be[...], 0.0)
        mp_o[...] = jnp.dot(h0, wg[...], preferred_element_type=jnp.float32) * dis[:, None]
        dis_o[...] = dis[:, None]

    return pl.pallas_call(
        body,
        grid=(GRID,),
        in_specs=[
            pl.BlockSpec((BLK, D_IN), lambda i: (i, 0)),
            pl.BlockSpec((2, BLK), lambda i: (0, i)),
            pl.BlockSpec((D_IN, 64), lambda i: (0, 0)),
            pl.BlockSpec((64,), lambda i: (0,)),
            pl.BlockSpec((64, 32), lambda i: (0, 0)),
        ],
        out_specs=[
            pl.BlockSpec((BLK, 32), lambda i: (i, 0)),
            pl.BlockSpec((BLK, 1), lambda i: (i, 0)),
        ],
        out_shape=[
            jax.ShapeDtypeStruct((NP, 32), jnp.float32),
            jax.ShapeDtypeStruct((NP, 1), jnp.float32),
        ],
    )(x, degp2, W_enc, b_enc, Wg1)


def _tc_mid(accp, mp, dis, bg, Wn, Fi, Fo):
    def body(ab, mb, db, bgb, wb, o):
        d = db[...]
        h = jnp.maximum(d * (ab[0] + ab[1] + mb[...]) + bgb[...], 0.0)
        o[...] = jnp.dot(h, wb[...], preferred_element_type=jnp.float32) * d

    return pl.pallas_call(
        body,
        grid=(GRID,),
        in_specs=[
            pl.BlockSpec((2, BLK, Fi), lambda i: (0, i, 0)),
            pl.BlockSpec((BLK, Fi), lambda i: (i, 0)),
            pl.BlockSpec((BLK, 1), lambda i: (i, 0)),
            pl.BlockSpec((Fi,), lambda i: (0,)),
            pl.BlockSpec((Fi, Fo), lambda i: (0, 0)),
        ],
        out_specs=pl.BlockSpec((BLK, Fo), lambda i: (i, 0)),
        out_shape=jax.ShapeDtypeStruct((NP, Fo), jnp.float32),
    )(accp, mp, dis, bg, Wn)


def _tc_heads(accp, mp, dis, bg3, Ws1, bs1, Wv1, bv1, Wv2, bv2):
    def body(ab, mb, db, bgb, ws1, bs1b, wv1, bv1b, wv2, bv2b, a_o, b_o, vsq_o):
        d = db[...]
        h3 = d * (ab[0] + ab[1] + mb[...]) + bgb[...]
        ws1v = ws1[...]
        a_o[...] = jnp.dot(h3, ws1v[:8], preferred_element_type=jnp.float32) + bs1b[...]
        b_o[...] = jnp.dot(h3, ws1v[8:], preferred_element_type=jnp.float32)
        z = jnp.maximum(jnp.dot(h3, wv1[...], preferred_element_type=jnp.float32)
                        + bv1b[...], 0.0)
        t = jnp.dot(z, wv2[...], preferred_element_type=jnp.float32) + bv2b[...]
        vs = 1.0 / (1.0 + jnp.exp(-t))
        vp = 0.9 + 0.2 * vs
        vsq_o[...] = vp * vp

    return pl.pallas_call(
        body,
        grid=(GRID,),
        in_specs=[
            pl.BlockSpec((2, BLK, 8), lambda i: (0, i, 0)),
            pl.BlockSpec((BLK, 8), lambda i: (i, 0)),
            pl.BlockSpec((BLK, 1), lambda i: (i, 0)),
            pl.BlockSpec((8,), lambda i: (0,)),
            pl.BlockSpec((16, 8), lambda i: (0, 0)),
            pl.BlockSpec((8,), lambda i: (0,)),
            pl.BlockSpec((8, 4), lambda i: (0, 0)),
            pl.BlockSpec((4,), lambda i: (0,)),
            pl.BlockSpec((4, 1), lambda i: (0, 0)),
            pl.BlockSpec((1,), lambda i: (0,)),
        ],
        out_specs=[
            pl.BlockSpec((BLK, 8), lambda i: (i, 0)),
            pl.BlockSpec((BLK, 8), lambda i: (i, 0)),
            pl.BlockSpec((BLK, 1), lambda i: (i, 0)),
        ],
        out_shape=[
            jax.ShapeDtypeStruct((NP, 8), jnp.float32),
            jax.ShapeDtypeStruct((NP, 8), jnp.float32),
            jax.ShapeDtypeStruct((NP, 1), jnp.float32),
        ],
    )(accp, mp, dis, bg3, Ws1, bs1, Wv1, bv1, Wv2, bv2)


def kernel(x, edge_index, W_enc, b_enc, Wg1, bg1, Wg2, bg2, Wg3, bg3,
           Ws1, bs1, Ws2, bs2, Wv1, bv1, Wv2, bv2):
    ei = edge_index.astype(jnp.int32)

    degp = _deg_partials(ei, jnp.zeros((CHUNK,), jnp.float32)).reshape(2, NP)
    mp1, dis = _tc_encode(x, degp, W_enc, b_enc, Wg1)

    acc1 = _seg_sum_partials(mp1, ei, jnp.zeros((CHUNK, 32), jnp.float32),
                             32, 7).reshape(2, NP, 32)
    mp2 = _tc_mid(acc1, mp1, dis, bg1, Wg2, 32, 16)

    acc2 = _seg_sum_partials(mp2, ei, jnp.zeros((CHUNK, 16), jnp.float32),
                             16, 28).reshape(2, NP, 16)
    mp3 = _tc_mid(acc2, mp2, dis, bg2, Wg3, 16, 8)

    acc3 = _seg_sum_partials(mp3, ei, jnp.zeros((CHUNK, 8), jnp.float32),
                             8, 28).reshape(2, NP, 8)
    A, B, vsq = _tc_heads(acc3, mp3, dis, bg3, Ws1, bs1, Wv1, bv1, Wv2, bv2)

    w2s = jnp.tile(Ws2.reshape(8, 1), (1, 16))
    b2s = jnp.full((16,), 1.0, jnp.float32) * bs2[0]
    sw = _edge_head(A, B, ei, w2s, b2s)

    return sw, vsq[:N, 0]


# speedup vs baseline: 44.4604x; 1.1005x over previous
import functools

import jax
import jax.numpy as jnp
from jax import lax
from jax.experimental import pallas as pl
from jax.experimental.pallas import tpu as pltpu
from jax.experimental.pallas import tpu_sc as plsc

N = 50000
E = 800000
D_IN = 128

NCORES = 2
NSUB = 16
NW = NCORES * NSUB
CHUNK = 128
EPT = E // NW
NFC = EPT // CHUNK
TAIL = EPT - NFC * CHUNK
K = 28
NP = 50176
STRIPE = NP // NSUB

BLK = 6272
GRID = NP // BLK

SFULL = STRIPE // CHUNK
STAIL = STRIPE - SFULL * CHUNK
SCH = SFULL + 1


def _mesh():
    return plsc.VectorSubcoreMesh(
        core_axis_name="c", subcore_axis_name="s",
        num_cores=NCORES, num_subcores=NSUB)


def _zero_stripe(zrow, acc, r0, sem):
    cps = []
    for t in range(SFULL):
        off = pl.multiple_of(r0 + t * CHUNK, 8)
        cps.append(pltpu.async_copy(zrow, acc.at[pl.ds(off, CHUNK)], sem))
    cps.append(pltpu.async_copy(
        zrow.at[pl.ds(0, STAIL)],
        acc.at[pl.ds(r0 + SFULL * CHUNK, STAIL)], sem))
    for cp in cps:
        cp.wait()


def _write_stripe(buf, nslots, acc, out_hbm, r0, obase, sem_l, sem_w):
    chunks = [(t, CHUNK) for t in range(SFULL)] + [(SFULL, STAIL)]
    for w0 in range(0, SCH, nslots):
        wave = chunks[w0:w0 + nslots]
        cps = []
        for slot, (t, ln) in enumerate(wave):
            off = pl.multiple_of(r0 + t * CHUNK, 8)
            cps.append(pltpu.async_copy(acc.at[pl.ds(off, ln)],
                                        buf.at[slot, pl.ds(0, ln)], sem_l))
        for cp in cps:
            cp.wait()
        cps = []
        for slot, (t, ln) in enumerate(wave):
            off = pl.multiple_of(r0 + t * CHUNK, 8)
            cps.append(pltpu.async_copy(buf.at[slot, pl.ds(0, ln)],
                                        out_hbm.at[pl.ds(obase + off, ln)], sem_w))
        for cp in cps:
            cp.wait()


def _deg_partials(ei, zrow):
    @functools.partial(
        pl.kernel,
        out_type=jax.ShapeDtypeStruct((2 * NP,), jnp.float32),
        mesh=_mesh(),
        compiler_params=pltpu.CompilerParams(
            use_tc_tiling_on_sc=False, needs_layout_passes=False),
        scratch_types=[
            pltpu.VMEM((K, CHUNK), jnp.int32),
            pltpu.VMEM((TAIL,), jnp.int32),
            pltpu.VMEM((CHUNK,), jnp.float32),
            pltpu.VMEM((SCH, CHUNK), jnp.float32),
            pltpu.VMEM_SHARED((NP,), jnp.float32),
            pltpu.SemaphoreType.DMA,
            pltpu.SemaphoreType.DMA,
        ],
    )
    def k(ei_hbm, zero_hbm, out_hbm, gdidx, tdidx, ones, buf, acc, sem_i, sem_s):
        c = lax.axis_index("c")
        s = lax.axis_index("s")
        wid = c * NSUB + s
        r0 = pl.multiple_of(s * STRIPE, 8)
        pltpu.sync_copy(zero_hbm, ones)
        _zero_stripe(ones, acc, r0, sem_i)
        plsc.subcore_barrier()
        for i in range(CHUNK // 16):
            ones[pl.ds(16 * i, 16)] = jnp.full((16,), 1.0, jnp.float32)
        ebase = wid * EPT

        def do_group(cbase, nch):
            eb = pl.multiple_of(ebase + cbase * CHUNK, 8)
            cps = [pltpu.async_copy(
                       ei_hbm.at[1, pl.ds(eb + b * CHUNK, CHUNK)],
                       gdidx.at[b], sem_i)
                   for b in range(nch)]
            for cp in cps:
                cp.wait()
            cps = [pltpu.async_copy(ones, acc.at[gdidx.at[b]], sem_s, add=True)
                   for b in range(nch)]
            for cp in cps:
                cp.wait()

        def body(g, carry):
            do_group(g * K, K)
            return carry

        gf = NFC // K
        lax.fori_loop(0, gf, body, 0)
        if NFC % K:
            do_group(gf * K, NFC % K)
        teb = pl.multiple_of(ebase + NFC * CHUNK, 8)
        pltpu.sync_copy(ei_hbm.at[1, pl.ds(teb, TAIL)], tdidx)
        pltpu.sync_copy(ones.at[pl.ds(0, TAIL)], acc.at[tdidx], add=True)
        plsc.subcore_barrier()
        _write_stripe(buf, SCH, acc, out_hbm, r0, c * NP, sem_i, sem_s)

    return k(ei, zrow)


def _seg_sum_partials(mp, ei, zrow, F, kc):
    @functools.partial(
        pl.kernel,
        out_type=jax.ShapeDtypeStruct((2 * NP, F), jnp.float32),
        mesh=_mesh(),
        compiler_params=pltpu.CompilerParams(
            use_tc_tiling_on_sc=False, needs_layout_passes=False),
        scratch_types=[
            pltpu.VMEM((kc * CHUNK,), jnp.int32),
            pltpu.VMEM((kc, CHUNK), jnp.int32),
            pltpu.VMEM((TAIL,), jnp.int32),
            pltpu.VMEM((TAIL,), jnp.int32),
            pltpu.VMEM((kc, CHUNK, F), jnp.float32),
            pltpu.VMEM_SHARED((NP, F), jnp.float32),
            pltpu.SemaphoreType.DMA,
            pltpu.SemaphoreType.DMA,
            pltpu.SemaphoreType.DMA,
        ],
    )
    def k(mp_hbm, ei_hbm, zero_hbm, out_hbm,
          gsidx, gdidx, tsidx, tdidx, rows, acc, sem_i, sem_g, sem_s):
        c = lax.axis_index("c")
        s = lax.axis_index("s")
        wid = c * NSUB + s
        r0 = pl.multiple_of(s * STRIPE, 8)
        pltpu.sync_copy(zero_hbm, rows.at[0])
        _zero_stripe(rows.at[0], acc, r0, sem_i)
        plsc.subcore_barrier()
        ebase = wid * EPT

        def do_group(cbase, nch):
            eb = pl.multiple_of(ebase + cbase * CHUNK, 8)
            cps = [pltpu.async_copy(ei_hbm.at[0, pl.ds(eb, nch * CHUNK)],
                                    gsidx.at[pl.ds(0, nch * CHUNK)], sem_i)]
            cps += [pltpu.async_copy(
                        ei_hbm.at[1, pl.ds(eb + b * CHUNK, CHUNK)],
                        gdidx.at[b], sem_i)
                    for b in range(nch)]
            for cp in cps:
                cp.wait()
            gcps = [pltpu.async_copy(
                        mp_hbm.at[gsidx.at[pl.ds(b * CHUNK, CHUNK)]],
                        rows.at[b], sem_g)
                    for b in range(nch)]
            scps = []
            for b in range(nch):
                gcps[b].wait()
                scps.append(pltpu.async_copy(rows.at[b], acc.at[gdidx.at[b]],
                                             sem_s, add=True))
            for cp in scps:
                cp.wait()

        def body(g, carry):
            do_group(g * kc, kc)
            return carry

        gf = NFC // kc
        lax.fori_loop(0, gf, body, 0)
        if NFC % kc:
            do_group(gf * kc, NFC % kc)
        teb = pl.multiple_of(ebase + NFC * CHUNK, 8)
        pltpu.sync_copy(ei_hbm.at[0, pl.ds(teb, TAIL)], tsidx)
        pltpu.sync_copy(ei_hbm.at[1, pl.ds(teb, TAIL)], tdidx)
        pltpu.async_copy(mp_hbm.at[tsidx], rows.at[0, pl.ds(0, TAIL)], sem_g).wait()
        pltpu.sync_copy(rows.at[0, pl.ds(0, TAIL)], acc.at[tdidx], add=True)
        plsc.subcore_barrier()
        _write_stripe(rows, kc, acc, out_hbm, r0, c * NP, sem_g, sem_s)

    return k(mp, ei, zrow)


def _edge_head(A, B, ei, w2s, b2s):
    @functools.partial(
        pl.kernel,
        out_type=jax.ShapeDtypeStruct((E,), jnp.float32),
        mesh=_mesh(),
        compiler_params=pltpu.CompilerParams(
            use_tc_tiling_on_sc=False, needs_layout_passes=False),
        scratch_types=[
            pltpu.VMEM((K * CHUNK,), jnp.int32),
            pltpu.VMEM((K * CHUNK,), jnp.int32),
            pltpu.VMEM((TAIL,), jnp.int32),
            pltpu.VMEM((TAIL,), jnp.int32),
            pltpu.VMEM((K, CHUNK, 8), jnp.float32),
            pltpu.VMEM((K, CHUNK, 8), jnp.float32),
            pltpu.VMEM((8, 16), jnp.float32),
            pltpu.VMEM((16,), jnp.float32),
            pltpu.VMEM((K * CHUNK,), jnp.float32),
            pltpu.SemaphoreType.DMA,
            pltpu.SemaphoreType.DMA,
            pltpu.SemaphoreType.DMA,
        ],
    )
    def k(a_hbm, b_hbm, ei_hbm, w_hbm, b2_hbm, out_hbm,
          gsidx, gdidx, tsidx, tdidx, ra, rb, wv, b2v, ob, sem_i, sem_g, sem_w):
        c = lax.axis_index("c")
        s = lax.axis_index("s")
        wid = c * NSUB + s
        pltpu.sync_copy(w_hbm, wv)
        pltpu.sync_copy(b2_hbm, b2v)
        wsp = [wv[j, :] for j in range(8)]
        b2 = b2v[...]
        iota = lax.iota(jnp.int32, 16)
        ebase = wid * EPT

        def compute_chunk(b, nvec):
            bb = jnp.full((16,), b, jnp.int32)
            for gg in range(nvec):
                ridx = iota + (gg * 16)
                acc = jnp.zeros((16,), jnp.float32)
                for j in range(8):
                    cj = jnp.full((16,), j, jnp.int32)
                    av = plsc.load_gather(ra, [bb, ridx, cj])
                    bv = plsc.load_gather(rb, [bb, ridx, cj])
                    acc = acc + jnp.maximum(av + bv, 0.0) * wsp[j]
                t = acc + b2
                ob[pl.ds(b * CHUNK + gg * 16, 16)] = 1.0 / (1.0 + jnp.exp(-t))

        def do_group(cbase, nch):
            eb = pl.multiple_of(ebase + cbase * CHUNK, 8)
            nw = nch * CHUNK
            cpi = pltpu.async_copy(ei_hbm.at[0, pl.ds(eb, nw)],
                                   gsidx.at[pl.ds(0, nw)], sem_i)
            cpd = pltpu.async_copy(ei_hbm.at[1, pl.ds(eb, nw)],
                                   gdidx.at[pl.ds(0, nw)], sem_i)
            cpi.wait()
            cpd.wait()
            for b in range(nch):
                sl = pl.ds(b * CHUNK, CHUNK)
                pltpu.async_copy(a_hbm.at[gsidx.at[sl]], ra.at[b], sem_g)
                pltpu.async_copy(b_hbm.at[gdidx.at[sl]], rb.at[b], sem_g)

            def chunk(b, carry2):
                pltpu.make_async_copy(a_hbm.at[pl.ds(0, CHUNK)], ra.at[0], sem_g).wait()
                pltpu.make_async_copy(b_hbm.at[pl.ds(0, CHUNK)], rb.at[0], sem_g).wait()
                compute_chunk(b, CHUNK // 16)
                return carry2

            lax.fori_loop(0, nch, chunk, 0)
            pltpu.async_copy(ob.at[pl.ds(0, nw)],
                             out_hbm.at[pl.ds(eb, nw)], sem_w).wait()

        def body(g, carry):
            do_group(g * K, K)
            return carry

        gf = NFC // K
        lax.fori_loop(0, gf, body, 0)
        if NFC % K:
            do_group(gf * K, NFC % K)
        teb = pl.multiple_of(ebase + NFC * CHUNK, 8)
        pltpu.sync_copy(ei_hbm.at[0, pl.ds(teb, TAIL)], tsidx)
        pltpu.sync_copy(ei_hbm.at[1, pl.ds(teb, TAIL)], tdidx)
        cpa = pltpu.async_copy(a_hbm.at[tsidx], ra.at[0, pl.ds(0, TAIL)], sem_g)
        cpb = pltpu.async_copy(b_hbm.at[tdidx], rb.at[0, pl.ds(0, TAIL)], sem_g)
        cpa.wait()
        cpb.wait()
        compute_chunk(0, (TAIL + 15) // 16)
        pltpu.sync_copy(ob.at[pl.ds(0, TAIL)], out_hbm.at[pl.ds(teb, TAIL)])

    return k(A, B, ei, w2s, b2s)


def _tc_encode(x, degp2, W_enc, b_enc, Wg1):
    def body(xb, degb, we, be, wg, mp_o, dis_o):
        deg = degb[0] + degb[1] + 1.0
        dis = lax.rsqrt(deg)
        h0 = jnp.maximum(jnp.dot(xb[...], we[...],
                                 preferred_element_type=jnp.float32) + be[...], 0.0)
        mp_o[...] = jnp.dot(h0, wg[...], preferred_element_type=jnp.float32) * dis[:, None]
        dis_o[...] = dis[:, None]

    return pl.pallas_call(
        body,
        grid=(GRID,),
        in_specs=[
            pl.BlockSpec((BLK, D_IN), lambda i: (i, 0)),
            pl.BlockSpec((2, BLK), lambda i: (0, i)),
            pl.BlockSpec((D_IN, 64), lambda i: (0, 0)),
            pl.BlockSpec((64,), lambda i: (0,)),
            pl.BlockSpec((64, 32), lambda i: (0, 0)),
        ],
        out_specs=[
            pl.BlockSpec((BLK, 32), lambda i: (i, 0)),
            pl.BlockSpec((BLK, 1), lambda i: (i, 0)),
        ],
        out_shape=[
            jax.ShapeDtypeStruct((NP, 32), jnp.float32),
            jax.ShapeDtypeStruct((NP, 1), jnp.float32),
        ],
    )(x, degp2, W_enc, b_enc, Wg1)


def _tc_mid(accp, mp, dis, bg, Wn, Fi, Fo):
    def body(ab, mb, db, bgb, wb, o):
        d = db[...]
        h = jnp.maximum(d * (ab[0] + ab[1] + mb[...]) + bgb[...], 0.0)
        o[...] = jnp.dot(h, wb[...], preferred_element_type=jnp.float32) * d

    return pl.pallas_call(
        body,
        grid=(GRID,),
        in_specs=[
            pl.BlockSpec((2, BLK, Fi), lambda i: (0, i, 0)),
            pl.BlockSpec((BLK, Fi), lambda i: (i, 0)),
            pl.BlockSpec((BLK, 1), lambda i: (i, 0)),
            pl.BlockSpec((Fi,), lambda i: (0,)),
            pl.BlockSpec((Fi, Fo), lambda i: (0, 0)),
        ],
        out_specs=pl.BlockSpec((BLK, Fo), lambda i: (i, 0)),
        out_shape=jax.ShapeDtypeStruct((NP, Fo), jnp.float32),
    )(accp, mp, dis, bg, Wn)


def _tc_heads(accp, mp, dis, bg3, Ws1, bs1, Wv1, bv1, Wv2, bv2):
    def body(ab, mb, db, bgb, ws1, bs1b, wv1, bv1b, wv2, bv2b, a_o, b_o, vsq_o):
        d = db[...]
        h3 = d * (ab[0] + ab[1] + mb[...]) + bgb[...]
        ws1v = ws1[...]
        a_o[...] = jnp.dot(h3, ws1v[:8], preferred_element_type=jnp.float32) + bs1b[...]
        b_o[...] = jnp.dot(h3, ws1v[8:], preferred_element_type=jnp.float32)
        z = jnp.maximum(jnp.dot(h3, wv1[...], preferred_element_type=jnp.float32)
                        + bv1b[...], 0.0)
        t = jnp.dot(z, wv2[...], preferred_element_type=jnp.float32) + bv2b[...]
        vs = 1.0 / (1.0 + jnp.exp(-t))
        vp = 0.9 + 0.2 * vs
        vsq_o[...] = vp * vp

    return pl.pallas_call(
        body,
        grid=(GRID,),
        in_specs=[
            pl.BlockSpec((2, BLK, 8), lambda i: (0, i, 0)),
            pl.BlockSpec((BLK, 8), lambda i: (i, 0)),
            pl.BlockSpec((BLK, 1), lambda i: (i, 0)),
            pl.BlockSpec((8,), lambda i: (0,)),
            pl.BlockSpec((16, 8), lambda i: (0, 0)),
            pl.BlockSpec((8,), lambda i: (0,)),
            pl.BlockSpec((8, 4), lambda i: (0, 0)),
            pl.BlockSpec((4,), lambda i: (0,)),
            pl.BlockSpec((4, 1), lambda i: (0, 0)),
            pl.BlockSpec((1,), lambda i: (0,)),
        ],
        out_specs=[
            pl.BlockSpec((BLK, 8), lambda i: (i, 0)),
            pl.BlockSpec((BLK, 8), lambda i: (i, 0)),
            pl.BlockSpec((BLK, 1), lambda i: (i, 0)),
        ],
        out_shape=[
            jax.ShapeDtypeStruct((NP, 8), jnp.float32),
            jax.ShapeDtypeStruct((NP, 8), jnp.float32),
            jax.ShapeDtypeStruct((NP, 1), jnp.float32),
        ],
    )(accp, mp, dis, bg3, Ws1, bs1, Wv1, bv1, Wv2, bv2)


def kernel(x, edge_index, W_enc, b_enc, Wg1, bg1, Wg2, bg2, Wg3, bg3,
           Ws1, bs1, Ws2, bs2, Wv1, bv1, Wv2, bv2):
    ei = edge_index.astype(jnp.int32)

    degp = _deg_partials(ei, jnp.zeros((CHUNK,), jnp.float32)).reshape(2, NP)
    mp1, dis = _tc_encode(x, degp, W_enc, b_enc, Wg1)

    acc1 = _seg_sum_partials(mp1, ei, jnp.zeros((CHUNK, 32), jnp.float32),
                             32, 7).reshape(2, NP, 32)
    mp2 = _tc_mid(acc1, mp1, dis, bg1, Wg2, 32, 16)

    acc2 = _seg_sum_partials(mp2, ei, jnp.zeros((CHUNK, 16), jnp.float32),
                             16, 28).reshape(2, NP, 16)
    mp3 = _tc_mid(acc2, mp2, dis, bg2, Wg3, 16, 8)

    acc3 = _seg_sum_partials(mp3, ei, jnp.zeros((CHUNK, 8), jnp.float32),
                             8, 28).reshape(2, NP, 8)
    A, B, vsq = _tc_heads(acc3, mp3, dis, bg3, Ws1, bs1, Wv1, bv1, Wv2, bv2)

    w2s = jnp.tile(Ws2.reshape(8, 1), (1, 16))
    b2s = jnp.full((16,), 1.0, jnp.float32) * bs2[0]
    sw = _edge_head(A, B, ei, w2s, b2s)

    return sw, vsq[:N, 0]


# speedup vs baseline: 46.0339x; 1.0354x over previous
import functools

import jax
import jax.numpy as jnp
from jax import lax
from jax.experimental import pallas as pl
from jax.experimental.pallas import tpu as pltpu
from jax.experimental.pallas import tpu_sc as plsc

N = 50000
E = 800000
D_IN = 128

NCORES = 2
NSUB = 16
NW = NCORES * NSUB
CHUNK = 128
EPT = E // NW
NFC = EPT // CHUNK
TAIL = EPT - NFC * CHUNK
K = 28
NP = 50176
STRIPE = NP // NSUB

BLK = 6272
GRID = NP // BLK

SFULL = STRIPE // CHUNK
STAIL = STRIPE - SFULL * CHUNK
SCH = SFULL + 1


def _mesh():
    return plsc.VectorSubcoreMesh(
        core_axis_name="c", subcore_axis_name="s",
        num_cores=NCORES, num_subcores=NSUB)


def _zero_stripe(zrow, acc, r0, sem):
    cps = []
    for t in range(SFULL):
        off = pl.multiple_of(r0 + t * CHUNK, 8)
        cps.append(pltpu.async_copy(zrow, acc.at[pl.ds(off, CHUNK)], sem))
    cps.append(pltpu.async_copy(
        zrow.at[pl.ds(0, STAIL)],
        acc.at[pl.ds(r0 + SFULL * CHUNK, STAIL)], sem))
    for cp in cps:
        cp.wait()


def _write_stripe(buf, nslots, acc, out_hbm, r0, obase, sem_l, sem_w):
    chunks = [(t, CHUNK) for t in range(SFULL)] + [(SFULL, STAIL)]
    for w0 in range(0, SCH, nslots):
        wave = chunks[w0:w0 + nslots]
        cps = []
        for slot, (t, ln) in enumerate(wave):
            off = pl.multiple_of(r0 + t * CHUNK, 8)
            cps.append(pltpu.async_copy(acc.at[pl.ds(off, ln)],
                                        buf.at[slot, pl.ds(0, ln)], sem_l))
        for cp in cps:
            cp.wait()
        cps = []
        for slot, (t, ln) in enumerate(wave):
            off = pl.multiple_of(r0 + t * CHUNK, 8)
            cps.append(pltpu.async_copy(buf.at[slot, pl.ds(0, ln)],
                                        out_hbm.at[pl.ds(obase + off, ln)], sem_w))
        for cp in cps:
            cp.wait()


def _deg_partials(ei, zrow):
    @functools.partial(
        pl.kernel,
        out_type=jax.ShapeDtypeStruct((2 * NP,), jnp.float32),
        mesh=_mesh(),
        compiler_params=pltpu.CompilerParams(
            use_tc_tiling_on_sc=False, needs_layout_passes=False),
        scratch_types=[
            pltpu.VMEM((K, CHUNK), jnp.int32),
            pltpu.VMEM((TAIL,), jnp.int32),
            pltpu.VMEM((CHUNK,), jnp.float32),
            pltpu.VMEM((SCH, CHUNK), jnp.float32),
            pltpu.VMEM_SHARED((NP,), jnp.float32),
            pltpu.SemaphoreType.DMA,
            pltpu.SemaphoreType.DMA,
        ],
    )
    def k(ei_hbm, zero_hbm, out_hbm, gdidx, tdidx, ones, buf, acc, sem_i, sem_s):
        c = lax.axis_index("c")
        s = lax.axis_index("s")
        wid = c * NSUB + s
        r0 = pl.multiple_of(s * STRIPE, 8)
        pltpu.sync_copy(zero_hbm, ones)
        _zero_stripe(ones, acc, r0, sem_i)
        plsc.subcore_barrier()
        for i in range(CHUNK // 16):
            ones[pl.ds(16 * i, 16)] = jnp.full((16,), 1.0, jnp.float32)
        ebase = wid * EPT

        def do_group(cbase, nch):
            eb = pl.multiple_of(ebase + cbase * CHUNK, 8)
            cps = [pltpu.async_copy(
                       ei_hbm.at[1, pl.ds(eb + b * CHUNK, CHUNK)],
                       gdidx.at[b], sem_i)
                   for b in range(nch)]
            for cp in cps:
                cp.wait()
            cps = [pltpu.async_copy(ones, acc.at[gdidx.at[b]], sem_s, add=True)
                   for b in range(nch)]
            for cp in cps:
                cp.wait()

        def body(g, carry):
            do_group(g * K, K)
            return carry

        gf = NFC // K
        lax.fori_loop(0, gf, body, 0)
        if NFC % K:
            do_group(gf * K, NFC % K)
        teb = pl.multiple_of(ebase + NFC * CHUNK, 8)
        pltpu.sync_copy(ei_hbm.at[1, pl.ds(teb, TAIL)], tdidx)
        pltpu.sync_copy(ones.at[pl.ds(0, TAIL)], acc.at[tdidx], add=True)
        plsc.subcore_barrier()
        _write_stripe(buf, SCH, acc, out_hbm, r0, c * NP, sem_i, sem_s)

    return k(ei, zrow)


def _seg_sum_partials(mp, ei, zrow, F, kc):
    @functools.partial(
        pl.kernel,
        out_type=jax.ShapeDtypeStruct((2 * NP, F), jnp.float32),
        mesh=_mesh(),
        compiler_params=pltpu.CompilerParams(
            use_tc_tiling_on_sc=False, needs_layout_passes=False),
        scratch_types=[
            pltpu.VMEM((kc * CHUNK,), jnp.int32),
            pltpu.VMEM((kc, CHUNK), jnp.int32),
            pltpu.VMEM((TAIL,), jnp.int32),
            pltpu.VMEM((TAIL,), jnp.int32),
            pltpu.VMEM((kc, CHUNK, F), jnp.float32),
            pltpu.VMEM_SHARED((NP, F), jnp.float32),
            pltpu.SemaphoreType.DMA,
            pltpu.SemaphoreType.DMA,
            pltpu.SemaphoreType.DMA,
        ],
    )
    def k(mp_hbm, ei_hbm, zero_hbm, out_hbm,
          gsidx, gdidx, tsidx, tdidx, rows, acc, sem_i, sem_g, sem_s):
        c = lax.axis_index("c")
        s = lax.axis_index("s")
        wid = c * NSUB + s
        r0 = pl.multiple_of(s * STRIPE, 8)
        pltpu.sync_copy(zero_hbm, rows.at[0])
        _zero_stripe(rows.at[0], acc, r0, sem_i)
        plsc.subcore_barrier()
        ebase = wid * EPT

        def do_group(cbase, nch):
            eb = pl.multiple_of(ebase + cbase * CHUNK, 8)
            cps = [pltpu.async_copy(ei_hbm.at[0, pl.ds(eb, nch * CHUNK)],
                                    gsidx.at[pl.ds(0, nch * CHUNK)], sem_i)]
            cps += [pltpu.async_copy(
                        ei_hbm.at[1, pl.ds(eb + b * CHUNK, CHUNK)],
                        gdidx.at[b], sem_i)
                    for b in range(nch)]
            for cp in cps:
                cp.wait()
            gcps = [pltpu.async_copy(
                        mp_hbm.at[gsidx.at[pl.ds(b * CHUNK, CHUNK)]],
                        rows.at[b], sem_g)
                    for b in range(nch)]
            scps = []
            for b in range(nch):
                gcps[b].wait()
                scps.append(pltpu.async_copy(rows.at[b], acc.at[gdidx.at[b]],
                                             sem_s, add=True))
            for cp in scps:
                cp.wait()

        def body(g, carry):
            do_group(g * kc, kc)
            return carry

        gf = NFC // kc
        lax.fori_loop(0, gf, body, 0)
        if NFC % kc:
            do_group(gf * kc, NFC % kc)
        teb = pl.multiple_of(ebase + NFC * CHUNK, 8)
        pltpu.sync_copy(ei_hbm.at[0, pl.ds(teb, TAIL)], tsidx)
        pltpu.sync_copy(ei_hbm.at[1, pl.ds(teb, TAIL)], tdidx)
        pltpu.async_copy(mp_hbm.at[tsidx], rows.at[0, pl.ds(0, TAIL)], sem_g).wait()
        pltpu.sync_copy(rows.at[0, pl.ds(0, TAIL)], acc.at[tdidx], add=True)
        plsc.subcore_barrier()
        _write_stripe(rows, kc, acc, out_hbm, r0, c * NP, sem_g, sem_s)

    return k(mp, ei, zrow)


def _edge_head(A, B, ei, w2s, b2s):
    @functools.partial(
        pl.kernel,
        out_type=jax.ShapeDtypeStruct((E,), jnp.float32),
        mesh=_mesh(),
        compiler_params=pltpu.CompilerParams(
            use_tc_tiling_on_sc=False, needs_layout_passes=False),
        scratch_types=[
            pltpu.VMEM((K * CHUNK,), jnp.int32),
            pltpu.VMEM((K * CHUNK,), jnp.int32),
            pltpu.VMEM((TAIL,), jnp.int32),
            pltpu.VMEM((TAIL,), jnp.int32),
            pltpu.VMEM((K, CHUNK, 8), jnp.float32),
            pltpu.VMEM((K, CHUNK, 8), jnp.float32),
            pltpu.VMEM((8, 16), jnp.float32),
            pltpu.VMEM((16,), jnp.float32),
            pltpu.VMEM((K * CHUNK,), jnp.float32),
            pltpu.SemaphoreType.DMA,
            pltpu.SemaphoreType.DMA,
            pltpu.SemaphoreType.DMA,
        ],
    )
    def k(a_hbm, b_hbm, ei_hbm, w_hbm, b2_hbm, out_hbm,
          gsidx, gdidx, tsidx, tdidx, ra, rb, wv, b2v, ob, sem_i, sem_g, sem_w):
        c = lax.axis_index("c")
        s = lax.axis_index("s")
        wid = c * NSUB + s
        pltpu.sync_copy(w_hbm, wv)
        pltpu.sync_copy(b2_hbm, b2v)
        wsp = [wv[j, :] for j in range(8)]
        b2 = b2v[...]
        iota = lax.iota(jnp.int32, 16)
        ebase = wid * EPT

        def compute_chunk(b, nvec):
            bb = jnp.full((16,), b, jnp.int32)
            for gg in range(nvec):
                ridx = iota + (gg * 16)
                acc = jnp.zeros((16,), jnp.float32)
                for j in range(8):
                    cj = jnp.full((16,), j, jnp.int32)
                    av = plsc.load_gather(ra, [bb, ridx, cj])
                    bv = plsc.load_gather(rb, [bb, ridx, cj])
                    acc = acc + jnp.maximum(av + bv, 0.0) * wsp[j]
                t = acc + b2
                ob[pl.ds(b * CHUNK + gg * 16, 16)] = 1.0 / (1.0 + jnp.exp(-t))

        def do_group(cbase, nch):
            eb = pl.multiple_of(ebase + cbase * CHUNK, 8)
            nw = nch * CHUNK
            cpi = pltpu.async_copy(ei_hbm.at[0, pl.ds(eb, nw)],
                                   gsidx.at[pl.ds(0, nw)], sem_i)
            cpd = pltpu.async_copy(ei_hbm.at[1, pl.ds(eb, nw)],
                                   gdidx.at[pl.ds(0, nw)], sem_i)
            cpi.wait()
            cpd.wait()
            for b in range(nch):
                sl = pl.ds(b * CHUNK, CHUNK)
                pltpu.async_copy(a_hbm.at[gsidx.at[sl]], ra.at[b], sem_g)
                pltpu.async_copy(b_hbm.at[gdidx.at[sl]], rb.at[b], sem_g)

            def chunk(b, carry2):
                pltpu.make_async_copy(a_hbm.at[pl.ds(0, CHUNK)], ra.at[0], sem_g).wait()
                pltpu.make_async_copy(b_hbm.at[pl.ds(0, CHUNK)], rb.at[0], sem_g).wait()
                compute_chunk(b, CHUNK // 16)
                return carry2

            lax.fori_loop(0, nch, chunk, 0)
            pltpu.async_copy(ob.at[pl.ds(0, nw)],
                             out_hbm.at[pl.ds(eb, nw)], sem_w).wait()

        def body(g, carry):
            do_group(g * K, K)
            return carry

        gf = NFC // K
        lax.fori_loop(0, gf, body, 0)
        if NFC % K:
            do_group(gf * K, NFC % K)
        teb = pl.multiple_of(ebase + NFC * CHUNK, 8)
        pltpu.sync_copy(ei_hbm.at[0, pl.ds(teb, TAIL)], tsidx)
        pltpu.sync_copy(ei_hbm.at[1, pl.ds(teb, TAIL)], tdidx)
        cpa = pltpu.async_copy(a_hbm.at[tsidx], ra.at[0, pl.ds(0, TAIL)], sem_g)
        cpb = pltpu.async_copy(b_hbm.at[tdidx], rb.at[0, pl.ds(0, TAIL)], sem_g)
        cpa.wait()
        cpb.wait()
        compute_chunk(0, (TAIL + 15) // 16)
        pltpu.sync_copy(ob.at[pl.ds(0, TAIL)], out_hbm.at[pl.ds(teb, TAIL)])

    return k(A, B, ei, w2s, b2s)


def _tc_encode(x, degp2, W_enc, b_enc, Wg1):
    def body(xb, degb, we, be, wg, mp_o, dis_o):
        deg = degb[0] + degb[1] + 1.0
        dis = lax.rsqrt(deg)
        h0 = jnp.maximum(jnp.dot(xb[...], we[...],
                                 preferred_element_type=jnp.float32) + be[...], 0.0)
        mp_o[...] = jnp.dot(h0, wg[...], preferred_element_type=jnp.float32) * dis[:, None]
        dis_o[...] = dis[:, None]

    return pl.pallas_call(
        body,
        grid=(GRID,),
        in_specs=[
            pl.BlockSpec((BLK, D_IN), lambda i: (i, 0)),
            pl.BlockSpec((2, BLK), lambda i: (0, i)),
            pl.BlockSpec((D_IN, 64), lambda i: (0, 0)),
            pl.BlockSpec((64,), lambda i: (0,)),
            pl.BlockSpec((64, 32), lambda i: (0, 0)),
        ],
        out_specs=[
            pl.BlockSpec((BLK, 32), lambda i: (i, 0)),
            pl.BlockSpec((BLK, 1), lambda i: (i, 0)),
        ],
        out_shape=[
            jax.ShapeDtypeStruct((NP, 32), jnp.float32),
            jax.ShapeDtypeStruct((NP, 1), jnp.float32),
        ],
    )(x, degp2, W_enc, b_enc, Wg1)


def _unflatten(s, Fi):
    g = 128 // Fi
    parts = [s[:, j * Fi:(j + 1) * Fi] for j in range(g)]
    return jnp.stack(parts, axis=1).reshape(s.shape[0] * g, Fi)


def _tc_mid(accf, mp, dis, bg, Wn, Fi, Fo):
    RB = BLK * Fi // 128

    def body(a0, a1, mb, db, bgb, wb, o):
        d = db[...]
        s = _unflatten(a0[...] + a1[...], Fi)
        h = jnp.maximum(d * (s + mb[...]) + bgb[...], 0.0)
        o[...] = jnp.dot(h, wb[...], preferred_element_type=jnp.float32) * d

    return pl.pallas_call(
        body,
        grid=(GRID,),
        in_specs=[
            pl.BlockSpec((RB, 128), lambda i: (i, 0)),
            pl.BlockSpec((RB, 128), lambda i: (i + GRID, 0)),
            pl.BlockSpec((BLK, Fi), lambda i: (i, 0)),
            pl.BlockSpec((BLK, 1), lambda i: (i, 0)),
            pl.BlockSpec((Fi,), lambda i: (0,)),
            pl.BlockSpec((Fi, Fo), lambda i: (0, 0)),
        ],
        out_specs=pl.BlockSpec((BLK, Fo), lambda i: (i, 0)),
        out_shape=jax.ShapeDtypeStruct((NP, Fo), jnp.float32),
    )(accf, accf, mp, dis, bg, Wn)


def _tc_heads(accf, mp, dis, bg3, Ws1, bs1, Wv1, bv1, Wv2, bv2):
    RB = BLK * 8 // 128

    def body(a0, a1, mb, db, bgb, ws1, bs1b, wv1, bv1b, wv2, bv2b, a_o, b_o, vsq_o):
        d = db[...]
        s = _unflatten(a0[...] + a1[...], 8)
        h3 = d * (s + mb[...]) + bgb[...]
        ws1v = ws1[...]
        a_o[...] = jnp.dot(h3, ws1v[:8], preferred_element_type=jnp.float32) + bs1b[...]
        b_o[...] = jnp.dot(h3, ws1v[8:], preferred_element_type=jnp.float32)
        z = jnp.maximum(jnp.dot(h3, wv1[...], preferred_element_type=jnp.float32)
                        + bv1b[...], 0.0)
        t = jnp.dot(z, wv2[...], preferred_element_type=jnp.float32) + bv2b[...]
        vs = 1.0 / (1.0 + jnp.exp(-t))
        vp = 0.9 + 0.2 * vs
        vsq_o[...] = vp * vp

    return pl.pallas_call(
        body,
        grid=(GRID,),
        in_specs=[
            pl.BlockSpec((RB, 128), lambda i: (i, 0)),
            pl.BlockSpec((RB, 128), lambda i: (i + GRID, 0)),
            pl.BlockSpec((BLK, 8), lambda i: (i, 0)),
            pl.BlockSpec((BLK, 1), lambda i: (i, 0)),
            pl.BlockSpec((8,), lambda i: (0,)),
            pl.BlockSpec((16, 8), lambda i: (0, 0)),
            pl.BlockSpec((8,), lambda i: (0,)),
            pl.BlockSpec((8, 4), lambda i: (0, 0)),
            pl.BlockSpec((4,), lambda i: (0,)),
            pl.BlockSpec((4, 1), lambda i: (0, 0)),
            pl.BlockSpec((1,), lambda i: (0,)),
        ],
        out_specs=[
            pl.BlockSpec((BLK, 8), lambda i: (i, 0)),
            pl.BlockSpec((BLK, 8), lambda i: (i, 0)),
            pl.BlockSpec((BLK, 1), lambda i: (i, 0)),
        ],
        out_shape=[
            jax.ShapeDtypeStruct((NP, 8), jnp.float32),
            jax.ShapeDtypeStruct((NP, 8), jnp.float32),
            jax.ShapeDtypeStruct((NP, 1), jnp.float32),
        ],
    )(accf, accf, mp, dis, bg3, Ws1, bs1, Wv1, bv1, Wv2, bv2)


def kernel(x, edge_index, W_enc, b_enc, Wg1, bg1, Wg2, bg2, Wg3, bg3,
           Ws1, bs1, Ws2, bs2, Wv1, bv1, Wv2, bv2):
    ei = edge_index.astype(jnp.int32)

    degp = _deg_partials(ei, jnp.zeros((CHUNK,), jnp.float32)).reshape(2, NP)
    mp1, dis = _tc_encode(x, degp, W_enc, b_enc, Wg1)

    acc1 = _seg_sum_partials(mp1, ei, jnp.zeros((CHUNK, 32), jnp.float32),
                             32, 7).reshape(2 * NP * 32 // 128, 128)
    mp2 = _tc_mid(acc1, mp1, dis, bg1, Wg2, 32, 16)

    acc2 = _seg_sum_partials(mp2, ei, jnp.zeros((CHUNK, 16), jnp.float32),
                             16, 28).reshape(2 * NP * 16 // 128, 128)
    mp3 = _tc_mid(acc2, mp2, dis, bg2, Wg3, 16, 8)

    acc3 = _seg_sum_partials(mp3, ei, jnp.zeros((CHUNK, 8), jnp.float32),
                             8, 28).reshape(2 * NP * 8 // 128, 128)
    A, B, vsq = _tc_heads(acc3, mp3, dis, bg3, Ws1, bs1, Wv1, bv1, Wv2, bv2)

    w2s = jnp.tile(Ws2.reshape(8, 1), (1, 16))
    b2s = jnp.full((16,), 1.0, jnp.float32) * bs2[0]
    sw = _edge_head(A, B, ei, w2s, b2s)

    return sw, vsq[:N, 0]


# speedup vs baseline: 47.8931x; 1.0404x over previous
import functools

import jax
import jax.numpy as jnp
from jax import lax
from jax.experimental import pallas as pl
from jax.experimental.pallas import tpu as pltpu
from jax.experimental.pallas import tpu_sc as plsc

N = 50000
E = 800000
D_IN = 128

NCORES = 2
NSUB = 16
NW = NCORES * NSUB
CHUNK = 128
EPT = E // NW
NFC = EPT // CHUNK
TAIL = EPT - NFC * CHUNK
K = 28
NP = 50176
STRIPE = NP // NSUB

BLK = 6272
GRID = NP // BLK

SFULL = STRIPE // CHUNK
STAIL = STRIPE - SFULL * CHUNK
SCH = SFULL + 1


def _mesh():
    return plsc.VectorSubcoreMesh(
        core_axis_name="c", subcore_axis_name="s",
        num_cores=NCORES, num_subcores=NSUB)


def _zero_stripe(zrow, acc, r0, sem):
    cps = []
    for t in range(SFULL):
        off = pl.multiple_of(r0 + t * CHUNK, 8)
        cps.append(pltpu.async_copy(zrow, acc.at[pl.ds(off, CHUNK)], sem))
    cps.append(pltpu.async_copy(
        zrow.at[pl.ds(0, STAIL)],
        acc.at[pl.ds(r0 + SFULL * CHUNK, STAIL)], sem))
    for cp in cps:
        cp.wait()


def _write_stripe(buf, nslots, acc, out_hbm, r0, obase, sem_l, sem_w):
    chunks = [(t, CHUNK) for t in range(SFULL)] + [(SFULL, STAIL)]
    for w0 in range(0, SCH, nslots):
        wave = chunks[w0:w0 + nslots]
        cps = []
        for slot, (t, ln) in enumerate(wave):
            off = pl.multiple_of(r0 + t * CHUNK, 8)
            cps.append(pltpu.async_copy(acc.at[pl.ds(off, ln)],
                                        buf.at[slot, pl.ds(0, ln)], sem_l))
        for cp in cps:
            cp.wait()
        cps = []
        for slot, (t, ln) in enumerate(wave):
            off = pl.multiple_of(r0 + t * CHUNK, 8)
            cps.append(pltpu.async_copy(buf.at[slot, pl.ds(0, ln)],
                                        out_hbm.at[pl.ds(obase + off, ln)], sem_w))
        for cp in cps:
            cp.wait()


def _deg_partials(ei, zrow):
    @functools.partial(
        pl.kernel,
        out_type=jax.ShapeDtypeStruct((2 * NP,), jnp.float32),
        mesh=_mesh(),
        compiler_params=pltpu.CompilerParams(
            use_tc_tiling_on_sc=False, needs_layout_passes=False),
        scratch_types=[
            pltpu.VMEM((K, CHUNK), jnp.int32),
            pltpu.VMEM((TAIL,), jnp.int32),
            pltpu.VMEM((CHUNK,), jnp.float32),
            pltpu.VMEM((SCH, CHUNK), jnp.float32),
            pltpu.VMEM_SHARED((NP,), jnp.float32),
            pltpu.SemaphoreType.DMA,
            pltpu.SemaphoreType.DMA,
        ],
    )
    def k(ei_hbm, zero_hbm, out_hbm, gdidx, tdidx, ones, buf, acc, sem_i, sem_s):
        c = lax.axis_index("c")
        s = lax.axis_index("s")
        wid = c * NSUB + s
        r0 = pl.multiple_of(s * STRIPE, 8)
        pltpu.sync_copy(zero_hbm, ones)
        _zero_stripe(ones, acc, r0, sem_i)
        plsc.subcore_barrier()
        for i in range(CHUNK // 16):
            ones[pl.ds(16 * i, 16)] = jnp.full((16,), 1.0, jnp.float32)
        ebase = wid * EPT

        def do_group(cbase, nch):
            eb = pl.multiple_of(ebase + cbase * CHUNK, 8)
            cps = [pltpu.async_copy(
                       ei_hbm.at[1, pl.ds(eb + b * CHUNK, CHUNK)],
                       gdidx.at[b], sem_i)
                   for b in range(nch)]
            for cp in cps:
                cp.wait()
            cps = [pltpu.async_copy(ones, acc.at[gdidx.at[b]], sem_s, add=True)
                   for b in range(nch)]
            for cp in cps:
                cp.wait()

        def body(g, carry):
            do_group(g * K, K)
            return carry

        gf = NFC // K
        lax.fori_loop(0, gf, body, 0)
        if NFC % K:
            do_group(gf * K, NFC % K)
        teb = pl.multiple_of(ebase + NFC * CHUNK, 8)
        pltpu.sync_copy(ei_hbm.at[1, pl.ds(teb, TAIL)], tdidx)
        pltpu.sync_copy(ones.at[pl.ds(0, TAIL)], acc.at[tdidx], add=True)
        plsc.subcore_barrier()
        _write_stripe(buf, SCH, acc, out_hbm, r0, c * NP, sem_i, sem_s)

    return k(ei, zrow)


def _seg_sum_partials(mp, ei, zrow, F, kc):
    @functools.partial(
        pl.kernel,
        out_type=jax.ShapeDtypeStruct((2 * NP, F), jnp.float32),
        mesh=_mesh(),
        compiler_params=pltpu.CompilerParams(
            use_tc_tiling_on_sc=False, needs_layout_passes=False),
        scratch_types=[
            pltpu.VMEM((kc * CHUNK,), jnp.int32),
            pltpu.VMEM((kc, CHUNK), jnp.int32),
            pltpu.VMEM((TAIL,), jnp.int32),
            pltpu.VMEM((TAIL,), jnp.int32),
            pltpu.VMEM((kc, CHUNK, F), jnp.float32),
            pltpu.VMEM_SHARED((NP, F), jnp.float32),
            pltpu.SemaphoreType.DMA,
            pltpu.SemaphoreType.DMA,
            pltpu.SemaphoreType.DMA,
        ],
    )
    def k(mp_hbm, ei_hbm, zero_hbm, out_hbm,
          gsidx, gdidx, tsidx, tdidx, rows, acc, sem_i, sem_g, sem_s):
        c = lax.axis_index("c")
        s = lax.axis_index("s")
        wid = c * NSUB + s
        r0 = pl.multiple_of(s * STRIPE, 8)
        pltpu.sync_copy(zero_hbm, rows.at[0])
        _zero_stripe(rows.at[0], acc, r0, sem_i)
        plsc.subcore_barrier()
        ebase = wid * EPT

        def do_group(cbase, nch):
            eb = pl.multiple_of(ebase + cbase * CHUNK, 8)
            cps = [pltpu.async_copy(ei_hbm.at[0, pl.ds(eb, nch * CHUNK)],
                                    gsidx.at[pl.ds(0, nch * CHUNK)], sem_i)]
            cps += [pltpu.async_copy(
                        ei_hbm.at[1, pl.ds(eb + b * CHUNK, CHUNK)],
                        gdidx.at[b], sem_i)
                    for b in range(nch)]
            for cp in cps:
                cp.wait()
            gcps = [pltpu.async_copy(
                        mp_hbm.at[gsidx.at[pl.ds(b * CHUNK, CHUNK)]],
                        rows.at[b], sem_g)
                    for b in range(nch)]
            scps = []
            for b in range(nch):
                gcps[b].wait()
                scps.append(pltpu.async_copy(rows.at[b], acc.at[gdidx.at[b]],
                                             sem_s, add=True))
            for cp in scps:
                cp.wait()

        def body(g, carry):
            do_group(g * kc, kc)
            return carry

        gf = NFC // kc
        lax.fori_loop(0, gf, body, 0)
        if NFC % kc:
            do_group(gf * kc, NFC % kc)
        teb = pl.multiple_of(ebase + NFC * CHUNK, 8)
        pltpu.sync_copy(ei_hbm.at[0, pl.ds(teb, TAIL)], tsidx)
        pltpu.sync_copy(ei_hbm.at[1, pl.ds(teb, TAIL)], tdidx)
        pltpu.async_copy(mp_hbm.at[tsidx], rows.at[0, pl.ds(0, TAIL)], sem_g).wait()
        pltpu.sync_copy(rows.at[0, pl.ds(0, TAIL)], acc.at[tdidx], add=True)
        plsc.subcore_barrier()
        _write_stripe(rows, kc, acc, out_hbm, r0, c * NP, sem_g, sem_s)

    return k(mp, ei, zrow)


def _edge_head(A, B, ei, w2s, b2s):
    @functools.partial(
        pl.kernel,
        out_type=jax.ShapeDtypeStruct((E,), jnp.float32),
        mesh=_mesh(),
        compiler_params=pltpu.CompilerParams(
            use_tc_tiling_on_sc=False, needs_layout_passes=False),
        scratch_types=[
            pltpu.VMEM((K * CHUNK,), jnp.int32),
            pltpu.VMEM((K * CHUNK,), jnp.int32),
            pltpu.VMEM((TAIL,), jnp.int32),
            pltpu.VMEM((TAIL,), jnp.int32),
            pltpu.VMEM((K, CHUNK, 8), jnp.float32),
            pltpu.VMEM((K, CHUNK, 8), jnp.float32),
            pltpu.VMEM((8, 16), jnp.float32),
            pltpu.VMEM((16,), jnp.float32),
            pltpu.VMEM((K * CHUNK,), jnp.float32),
            pltpu.SemaphoreType.DMA,
            pltpu.SemaphoreType.DMA,
            pltpu.SemaphoreType.DMA,
        ],
    )
    def k(a_hbm, b_hbm, ei_hbm, w_hbm, b2_hbm, out_hbm,
          gsidx, gdidx, tsidx, tdidx, ra, rb, wv, b2v, ob, sem_i, sem_g, sem_w):
        c = lax.axis_index("c")
        s = lax.axis_index("s")
        wid = c * NSUB + s
        pltpu.sync_copy(w_hbm, wv)
        pltpu.sync_copy(b2_hbm, b2v)
        wsp = [wv[j, :] for j in range(8)]
        b2 = b2v[...]
        iota = lax.iota(jnp.int32, 16)
        ebase = wid * EPT

        def compute_chunk(b, nvec):
            bb = jnp.full((16,), b, jnp.int32)
            for gg in range(nvec):
                ridx = iota + (gg * 16)
                acc = jnp.zeros((16,), jnp.float32)
                for j in range(8):
                    cj = jnp.full((16,), j, jnp.int32)
                    av = plsc.load_gather(ra, [bb, ridx, cj])
                    bv = plsc.load_gather(rb, [bb, ridx, cj])
                    acc = acc + jnp.maximum(av + bv, 0.0) * wsp[j]
                t = acc + b2
                ob[pl.ds(b * CHUNK + gg * 16, 16)] = 1.0 / (1.0 + jnp.exp(-t))

        def do_group(cbase, nch):
            eb = pl.multiple_of(ebase + cbase * CHUNK, 8)
            nw = nch * CHUNK
            cpi = pltpu.async_copy(ei_hbm.at[0, pl.ds(eb, nw)],
                                   gsidx.at[pl.ds(0, nw)], sem_i)
            cpd = pltpu.async_copy(ei_hbm.at[1, pl.ds(eb, nw)],
                                   gdidx.at[pl.ds(0, nw)], sem_i)
            cpi.wait()
            cpd.wait()
            for b in range(nch):
                sl = pl.ds(b * CHUNK, CHUNK)
                pltpu.async_copy(a_hbm.at[gsidx.at[sl]], ra.at[b], sem_g)
                pltpu.async_copy(b_hbm.at[gdidx.at[sl]], rb.at[b], sem_g)

            def chunk(b, carry2):
                pltpu.make_async_copy(a_hbm.at[pl.ds(0, CHUNK)], ra.at[0], sem_g).wait()
                pltpu.make_async_copy(b_hbm.at[pl.ds(0, CHUNK)], rb.at[0], sem_g).wait()
                compute_chunk(b, CHUNK // 16)
                return carry2

            lax.fori_loop(0, nch, chunk, 0)
            pltpu.async_copy(ob.at[pl.ds(0, nw)],
                             out_hbm.at[pl.ds(eb, nw)], sem_w).wait()

        def body(g, carry):
            do_group(g * K, K)
            return carry

        gf = NFC // K
        lax.fori_loop(0, gf, body, 0)
        if NFC % K:
            do_group(gf * K, NFC % K)
        teb = pl.multiple_of(ebase + NFC * CHUNK, 8)
        pltpu.sync_copy(ei_hbm.at[0, pl.ds(teb, TAIL)], tsidx)
        pltpu.sync_copy(ei_hbm.at[1, pl.ds(teb, TAIL)], tdidx)
        cpa = pltpu.async_copy(a_hbm.at[tsidx], ra.at[0, pl.ds(0, TAIL)], sem_g)
        cpb = pltpu.async_copy(b_hbm.at[tdidx], rb.at[0, pl.ds(0, TAIL)], sem_g)
        cpa.wait()
        cpb.wait()
        compute_chunk(0, (TAIL + 15) // 16)
        pltpu.sync_copy(ob.at[pl.ds(0, TAIL)], out_hbm.at[pl.ds(teb, TAIL)])

    return k(A, B, ei, w2s, b2s)


def _tc_encode(x, degp2, W_enc, b_enc, Wg1):
    def body(xb, degb, we, be, wg, mp_o):
        deg = degb[0] + degb[1] + 1.0
        dis = lax.rsqrt(deg)
        h0 = jnp.maximum(jnp.dot(xb[...], we[...],
                                 preferred_element_type=jnp.float32) + be[...], 0.0)
        mp_o[...] = jnp.dot(h0, wg[...], preferred_element_type=jnp.float32) * dis[:, None]

    return pl.pallas_call(
        body,
        grid=(GRID,),
        in_specs=[
            pl.BlockSpec((BLK, D_IN), lambda i: (i, 0)),
            pl.BlockSpec((2, BLK), lambda i: (0, i)),
            pl.BlockSpec((D_IN, 64), lambda i: (0, 0)),
            pl.BlockSpec((64,), lambda i: (0,)),
            pl.BlockSpec((64, 32), lambda i: (0, 0)),
        ],
        out_specs=pl.BlockSpec((BLK, 32), lambda i: (i, 0)),
        out_shape=jax.ShapeDtypeStruct((NP, 32), jnp.float32),
    )(x, degp2, W_enc, b_enc, Wg1)


def _unflatten(s, Fi):
    g = 128 // Fi
    parts = [s[:, j * Fi:(j + 1) * Fi] for j in range(g)]
    return jnp.stack(parts, axis=1).reshape(s.shape[0] * g, Fi)


def _tc_mid(accf, mp, degp2, bg, Wn, Fi, Fo):
    RB = BLK * Fi // 128

    def body(a0, a1, mb, db, bgb, wb, o):
        d = lax.rsqrt(db[0] + db[1] + 1.0)[:, None]
        s = _unflatten(a0[...] + a1[...], Fi)
        h = jnp.maximum(d * (s + mb[...]) + bgb[...], 0.0)
        o[...] = jnp.dot(h, wb[...], preferred_element_type=jnp.float32) * d

    return pl.pallas_call(
        body,
        grid=(GRID,),
        in_specs=[
            pl.BlockSpec((RB, 128), lambda i: (i, 0)),
            pl.BlockSpec((RB, 128), lambda i: (i + GRID, 0)),
            pl.BlockSpec((BLK, Fi), lambda i: (i, 0)),
            pl.BlockSpec((2, BLK), lambda i: (0, i)),
            pl.BlockSpec((Fi,), lambda i: (0,)),
            pl.BlockSpec((Fi, Fo), lambda i: (0, 0)),
        ],
        out_specs=pl.BlockSpec((BLK, Fo), lambda i: (i, 0)),
        out_shape=jax.ShapeDtypeStruct((NP, Fo), jnp.float32),
    )(accf, accf, mp, degp2, bg, Wn)


def _tc_heads(accf, mp, degp2, bg3, Ws1, bs1, Wv1, bv1, Wv2, bv2):
    RB = BLK * 8 // 128

    def body(a0, a1, mb, db, bgb, ws1, bs1b, wv1, bv1b, wv2, bv2b, a_o, b_o, vsq_o):
        d = lax.rsqrt(db[0] + db[1] + 1.0)[:, None]
        s = _unflatten(a0[...] + a1[...], 8)
        h3 = d * (s + mb[...]) + bgb[...]
        ws1v = ws1[...]
        a_o[...] = jnp.dot(h3, ws1v[:8], preferred_element_type=jnp.float32) + bs1b[...]
        b_o[...] = jnp.dot(h3, ws1v[8:], preferred_element_type=jnp.float32)
        z = jnp.maximum(jnp.dot(h3, wv1[...], preferred_element_type=jnp.float32)
                        + bv1b[...], 0.0)
        t = jnp.dot(z, wv2[...], preferred_element_type=jnp.float32) + bv2b[...]
        vs = 1.0 / (1.0 + jnp.exp(-t))
        vp = 0.9 + 0.2 * vs
        vsq_o[...] = vp * vp

    return pl.pallas_call(
        body,
        grid=(GRID,),
        in_specs=[
            pl.BlockSpec((RB, 128), lambda i: (i, 0)),
            pl.BlockSpec((RB, 128), lambda i: (i + GRID, 0)),
            pl.BlockSpec((BLK, 8), lambda i: (i, 0)),
            pl.BlockSpec((2, BLK), lambda i: (0, i)),
            pl.BlockSpec((8,), lambda i: (0,)),
            pl.BlockSpec((16, 8), lambda i: (0, 0)),
            pl.BlockSpec((8,), lambda i: (0,)),
            pl.BlockSpec((8, 4), lambda i: (0, 0)),
            pl.BlockSpec((4,), lambda i: (0,)),
            pl.BlockSpec((4, 1), lambda i: (0, 0)),
            pl.BlockSpec((1,), lambda i: (0,)),
        ],
        out_specs=[
            pl.BlockSpec((BLK, 8), lambda i: (i, 0)),
            pl.BlockSpec((BLK, 8), lambda i: (i, 0)),
            pl.BlockSpec((BLK, 1), lambda i: (i, 0)),
        ],
        out_shape=[
            jax.ShapeDtypeStruct((NP, 8), jnp.float32),
            jax.ShapeDtypeStruct((NP, 8), jnp.float32),
            jax.ShapeDtypeStruct((NP, 1), jnp.float32),
        ],
    )(accf, accf, mp, degp2, bg3, Ws1, bs1, Wv1, bv1, Wv2, bv2)


def kernel(x, edge_index, W_enc, b_enc, Wg1, bg1, Wg2, bg2, Wg3, bg3,
           Ws1, bs1, Ws2, bs2, Wv1, bv1, Wv2, bv2):
    ei = edge_index.astype(jnp.int32)

    degp = _deg_partials(ei, jnp.zeros((CHUNK,), jnp.float32)).reshape(2, NP)
    mp1 = _tc_encode(x, degp, W_enc, b_enc, Wg1)

    acc1 = _seg_sum_partials(mp1, ei, jnp.zeros((CHUNK, 32), jnp.float32),
                             32, 7).reshape(2 * NP * 32 // 128, 128)
    mp2 = _tc_mid(acc1, mp1, degp, bg1, Wg2, 32, 16)

    acc2 = _seg_sum_partials(mp2, ei, jnp.zeros((CHUNK, 16), jnp.float32),
                             16, 28).reshape(2 * NP * 16 // 128, 128)
    mp3 = _tc_mid(acc2, mp2, degp, bg2, Wg3, 16, 8)

    acc3 = _seg_sum_partials(mp3, ei, jnp.zeros((CHUNK, 8), jnp.float32),
                             8, 28).reshape(2 * NP * 8 // 128, 128)
    A, B, vsq = _tc_heads(acc3, mp3, degp, bg3, Ws1, bs1, Wv1, bv1, Wv2, bv2)

    w2s = jnp.tile(Ws2.reshape(8, 1), (1, 16))
    b2s = jnp.full((16,), 1.0, jnp.float32) * bs2[0]
    sw = _edge_head(A, B, ei, w2s, b2s)

    return sw, vsq[:N, 0]


# speedup vs baseline: 48.0042x; 1.0023x over previous
import functools

import jax
import jax.numpy as jnp
from jax import lax
from jax.experimental import pallas as pl
from jax.experimental.pallas import tpu as pltpu
from jax.experimental.pallas import tpu_sc as plsc

N = 50000
E = 800000
D_IN = 128

NCORES = 2
NSUB = 16
NW = NCORES * NSUB
CHUNK = 128
EPT = E // NW
NFC = EPT // CHUNK
TAIL = EPT - NFC * CHUNK
K = 28
KE = 24
NP = 50176
STRIPE = NP // NSUB

BLK = 6272
GRID = NP // BLK

SFULL = STRIPE // CHUNK
STAIL = STRIPE - SFULL * CHUNK
SCH = SFULL + 1


def _mesh():
    return plsc.VectorSubcoreMesh(
        core_axis_name="c", subcore_axis_name="s",
        num_cores=NCORES, num_subcores=NSUB)


def _zero_stripe(zrow, acc, r0, sem):
    cps = []
    for t in range(SFULL):
        off = pl.multiple_of(r0 + t * CHUNK, 8)
        cps.append(pltpu.async_copy(zrow, acc.at[pl.ds(off, CHUNK)], sem))
    cps.append(pltpu.async_copy(
        zrow.at[pl.ds(0, STAIL)],
        acc.at[pl.ds(r0 + SFULL * CHUNK, STAIL)], sem))
    for cp in cps:
        cp.wait()


def _write_stripe(buf, nslots, acc, out_hbm, r0, obase, sem_l, sem_w):
    chunks = [(t, CHUNK) for t in range(SFULL)] + [(SFULL, STAIL)]
    for w0 in range(0, SCH, nslots):
        wave = chunks[w0:w0 + nslots]
        cps = []
        for slot, (t, ln) in enumerate(wave):
            off = pl.multiple_of(r0 + t * CHUNK, 8)
            cps.append(pltpu.async_copy(acc.at[pl.ds(off, ln)],
                                        buf.at[slot, pl.ds(0, ln)], sem_l))
        for cp in cps:
            cp.wait()
        cps = []
        for slot, (t, ln) in enumerate(wave):
            off = pl.multiple_of(r0 + t * CHUNK, 8)
            cps.append(pltpu.async_copy(buf.at[slot, pl.ds(0, ln)],
                                        out_hbm.at[pl.ds(obase + off, ln)], sem_w))
        for cp in cps:
            cp.wait()


def _deg_partials(ei, zrow):
    @functools.partial(
        pl.kernel,
        out_type=jax.ShapeDtypeStruct((2 * NP,), jnp.float32),
        mesh=_mesh(),
        compiler_params=pltpu.CompilerParams(
            use_tc_tiling_on_sc=False, needs_layout_passes=False),
        scratch_types=[
            pltpu.VMEM((K, CHUNK), jnp.int32),
            pltpu.VMEM((TAIL,), jnp.int32),
            pltpu.VMEM((CHUNK,), jnp.float32),
            pltpu.VMEM((SCH, CHUNK), jnp.float32),
            pltpu.VMEM_SHARED((NP,), jnp.float32),
            pltpu.SemaphoreType.DMA,
            pltpu.SemaphoreType.DMA,
        ],
    )
    def k(ei_hbm, zero_hbm, out_hbm, gdidx, tdidx, ones, buf, acc, sem_i, sem_s):
        c = lax.axis_index("c")
        s = lax.axis_index("s")
        wid = c * NSUB + s
        r0 = pl.multiple_of(s * STRIPE, 8)
        pltpu.sync_copy(zero_hbm, ones)
        _zero_stripe(ones, acc, r0, sem_i)
        plsc.subcore_barrier()
        for i in range(CHUNK // 16):
            ones[pl.ds(16 * i, 16)] = jnp.full((16,), 1.0, jnp.float32)
        ebase = wid * EPT

        def do_group(cbase, nch):
            eb = pl.multiple_of(ebase + cbase * CHUNK, 8)
            cps = [pltpu.async_copy(
                       ei_hbm.at[pl.ds(E + eb + b * CHUNK, CHUNK)],
                       gdidx.at[b], sem_i)
                   for b in range(nch)]
            for cp in cps:
                cp.wait()
            cps = [pltpu.async_copy(ones, acc.at[gdidx.at[b]], sem_s, add=True)
                   for b in range(nch)]
            for cp in cps:
                cp.wait()

        def body(g, carry):
            do_group(g * K, K)
            return carry

        gf = NFC // K
        lax.fori_loop(0, gf, body, 0)
        if NFC % K:
            do_group(gf * K, NFC % K)
        teb = pl.multiple_of(ebase + NFC * CHUNK, 8)
        pltpu.sync_copy(ei_hbm.at[pl.ds(E + teb, TAIL)], tdidx)
        pltpu.sync_copy(ones.at[pl.ds(0, TAIL)], acc.at[tdidx], add=True)
        plsc.subcore_barrier()
        _write_stripe(buf, SCH, acc, out_hbm, r0, c * NP, sem_i, sem_s)

    return k(ei, zrow)


def _seg_sum_partials(mp, ei, zrow, F, kc):
    @functools.partial(
        pl.kernel,
        out_type=jax.ShapeDtypeStruct((2 * NP, F), jnp.float32),
        mesh=_mesh(),
        compiler_params=pltpu.CompilerParams(
            use_tc_tiling_on_sc=False, needs_layout_passes=False),
        scratch_types=[
            pltpu.VMEM((kc * CHUNK,), jnp.int32),
            pltpu.VMEM((kc, CHUNK), jnp.int32),
            pltpu.VMEM((TAIL,), jnp.int32),
            pltpu.VMEM((TAIL,), jnp.int32),
            pltpu.VMEM((kc, CHUNK, F), jnp.float32),
            pltpu.VMEM_SHARED((NP, F), jnp.float32),
            pltpu.SemaphoreType.DMA,
            pltpu.SemaphoreType.DMA,
            pltpu.SemaphoreType.DMA,
        ],
    )
    def k(mp_hbm, ei_hbm, zero_hbm, out_hbm,
          gsidx, gdidx, tsidx, tdidx, rows, acc, sem_i, sem_g, sem_s):
        c = lax.axis_index("c")
        s = lax.axis_index("s")
        wid = c * NSUB + s
        r0 = pl.multiple_of(s * STRIPE, 8)
        pltpu.sync_copy(zero_hbm, rows.at[0])
        _zero_stripe(rows.at[0], acc, r0, sem_i)
        plsc.subcore_barrier()
        ebase = wid * EPT

        def do_group(cbase, nch):
            eb = pl.multiple_of(ebase + cbase * CHUNK, 8)
            cps = [pltpu.async_copy(ei_hbm.at[pl.ds(eb, nch * CHUNK)],
                                    gsidx.at[pl.ds(0, nch * CHUNK)], sem_i)]
            cps += [pltpu.async_copy(
                        ei_hbm.at[pl.ds(E + eb + b * CHUNK, CHUNK)],
                        gdidx.at[b], sem_i)
                    for b in range(nch)]
            for cp in cps:
                cp.wait()
            gcps = [pltpu.async_copy(
                        mp_hbm.at[gsidx.at[pl.ds(b * CHUNK, CHUNK)]],
                        rows.at[b], sem_g)
                    for b in range(nch)]
            scps = []
            for b in range(nch):
                gcps[b].wait()
                scps.append(pltpu.async_copy(rows.at[b], acc.at[gdidx.at[b]],
                                             sem_s, add=True))
            for cp in scps:
                cp.wait()

        def body(g, carry):
            do_group(g * kc, kc)
            return carry

        gf = NFC // kc
        lax.fori_loop(0, gf, body, 0)
        if NFC % kc:
            do_group(gf * kc, NFC % kc)
        teb = pl.multiple_of(ebase + NFC * CHUNK, 8)
        pltpu.sync_copy(ei_hbm.at[pl.ds(teb, TAIL)], tsidx)
        pltpu.sync_copy(ei_hbm.at[pl.ds(E + teb, TAIL)], tdidx)
        pltpu.async_copy(mp_hbm.at[tsidx], rows.at[0, pl.ds(0, TAIL)], sem_g).wait()
        pltpu.sync_copy(rows.at[0, pl.ds(0, TAIL)], acc.at[tdidx], add=True)
        plsc.subcore_barrier()
        _write_stripe(rows, kc, acc, out_hbm, r0, c * NP, sem_g, sem_s)

    return k(mp, ei, zrow)


def _edge_head(A, B, ei, w2s, b2s):
    @functools.partial(
        pl.kernel,
        out_type=jax.ShapeDtypeStruct((E,), jnp.float32),
        mesh=_mesh(),
        compiler_params=pltpu.CompilerParams(
            use_tc_tiling_on_sc=False, needs_layout_passes=False),
        scratch_types=[
            pltpu.VMEM((3, KE * CHUNK), jnp.int32),
            pltpu.VMEM((3, KE * CHUNK), jnp.int32),
            pltpu.VMEM((TAIL,), jnp.int32),
            pltpu.VMEM((TAIL,), jnp.int32),
            pltpu.VMEM((2, KE, CHUNK, 8), jnp.float32),
            pltpu.VMEM((2, KE, CHUNK, 8), jnp.float32),
            pltpu.VMEM((8, 16), jnp.float32),
            pltpu.VMEM((16,), jnp.float32),
            pltpu.VMEM((2, KE * CHUNK), jnp.float32),
            pltpu.SemaphoreType.DMA,
            pltpu.SemaphoreType.DMA,
            pltpu.SemaphoreType.DMA,
        ],
    )
    def k(a_hbm, b_hbm, ei_hbm, w_hbm, b2_hbm, out_hbm,
          gsidx, gdidx, tsidx, tdidx, ra, rb, wv, b2v, ob, sem_i, sem_g, sem_w):
        c = lax.axis_index("c")
        s = lax.axis_index("s")
        wid = c * NSUB + s
        pltpu.sync_copy(w_hbm, wv)
        pltpu.sync_copy(b2_hbm, b2v)
        wsp = [wv[j, :] for j in range(8)]
        b2 = b2v[...]
        iota = lax.iota(jnp.int32, 16)
        ebase = wid * EPT

        def compute_chunk(pb, b, nvec):
            bb = jnp.full((16,), b, jnp.int32)
            for gg in range(nvec):
                ridx = iota + (gg * 16)
                acc = jnp.zeros((16,), jnp.float32)
                for j in range(8):
                    cj = jnp.full((16,), j, jnp.int32)
                    av = plsc.load_gather(ra.at[pb], [bb, ridx, cj])
                    bv = plsc.load_gather(rb.at[pb], [bb, ridx, cj])
                    acc = acc + jnp.maximum(av + bv, 0.0) * wsp[j]
                t = acc + b2
                ob[pb, pl.ds(b * CHUNK + gg * 16, 16)] = 1.0 / (1.0 + jnp.exp(-t))

        grps = [(g * KE, KE) for g in range(NFC // KE)]
        if NFC % KE:
            grps.append(((NFC // KE) * KE, NFC % KE))
        ng = len(grps)

        def issue_idx(gi):
            cb, nch = grps[gi]
            qb = gi % 3
            eb = pl.multiple_of(ebase + cb * CHUNK, 8)
            nw = nch * CHUNK
            return (pltpu.async_copy(ei_hbm.at[pl.ds(eb, nw)],
                                     gsidx.at[qb, pl.ds(0, nw)], sem_i),
                    pltpu.async_copy(ei_hbm.at[pl.ds(E + eb, nw)],
                                     gdidx.at[qb, pl.ds(0, nw)], sem_i))

        def issue_gathers(gi):
            qb, pb, nch = gi % 3, gi % 2, grps[gi][1]
            for b in range(nch):
                sl = pl.ds(b * CHUNK, CHUNK)
                pltpu.async_copy(a_hbm.at[gsidx.at[qb, sl]], ra.at[pb, b], sem_g)
                pltpu.async_copy(b_hbm.at[gdidx.at[qb, sl]], rb.at[pb, b], sem_g)

        def compute_group(gi):
            pb, (cb, nch) = gi % 2, grps[gi]

            def chunk(b, carry2):
                pltpu.make_async_copy(a_hbm.at[pl.ds(0, CHUNK)],
                                      ra.at[0, 0], sem_g).wait()
                pltpu.make_async_copy(b_hbm.at[pl.ds(0, CHUNK)],
                                      rb.at[0, 0], sem_g).wait()
                compute_chunk(pb, b, CHUNK // 16)
                return carry2

            lax.fori_loop(0, nch, chunk, 0)
            eb = pl.multiple_of(ebase + cb * CHUNK, 8)
            nw = nch * CHUNK
            return pltpu.async_copy(ob.at[pb, pl.ds(0, nw)],
                                    out_hbm.at[pl.ds(eb, nw)], sem_w)

        icps = {0: issue_idx(0)}
        if ng > 1:
            icps[1] = issue_idx(1)
        stores = {}
        for gi in range(ng):
            icps[gi][0].wait()
            icps[gi][1].wait()
            issue_gathers(gi)
            if gi + 2 < ng:
                icps[gi + 2] = issue_idx(gi + 2)
            if gi % 2 in stores:
                stores[gi % 2].wait()
            stores[gi % 2] = compute_group(gi)
        teb = pl.multiple_of(ebase + NFC * CHUNK, 8)
        pltpu.sync_copy(ei_hbm.at[pl.ds(teb, TAIL)], tsidx)
        pltpu.sync_copy(ei_hbm.at[pl.ds(E + teb, TAIL)], tdidx)
        if 0 in stores:
            stores[0].wait()
        cpa = pltpu.async_copy(a_hbm.at[tsidx], ra.at[0, 0, pl.ds(0, TAIL)], sem_g)
        cpb = pltpu.async_copy(b_hbm.at[tdidx], rb.at[0, 0, pl.ds(0, TAIL)], sem_g)
        cpa.wait()
        cpb.wait()
        compute_chunk(0, 0, (TAIL + 15) // 16)
        pltpu.sync_copy(ob.at[0, pl.ds(0, TAIL)], out_hbm.at[pl.ds(teb, TAIL)])
        if 1 in stores:
            stores[1].wait()

    return k(A, B, ei, w2s, b2s)


def _tc_encode(x, degp2, W_enc, b_enc, Wg1):
    def body(xb, degb, we, be, wg, mp_o):
        deg = degb[0] + degb[1] + 1.0
        dis = lax.rsqrt(deg)
        h0 = jnp.maximum(jnp.dot(xb[...], we[...],
                                 preferred_element_type=jnp.float32) + be[...], 0.0)
        mp_o[...] = jnp.dot(h0, wg[...], preferred_element_type=jnp.float32) * dis[:, None]

    return pl.pallas_call(
        body,
        grid=(GRID,),
        in_specs=[
            pl.BlockSpec((BLK, D_IN), lambda i: (i, 0)),
            pl.BlockSpec((2, BLK), lambda i: (0, i)),
            pl.BlockSpec((D_IN, 64), lambda i: (0, 0)),
            pl.BlockSpec((64,), lambda i: (0,)),
            pl.BlockSpec((64, 32), lambda i: (0, 0)),
        ],
        out_specs=pl.BlockSpec((BLK, 32), lambda i: (i, 0)),
        out_shape=jax.ShapeDtypeStruct((NP, 32), jnp.float32),
    )(x, degp2, W_enc, b_enc, Wg1)


def _unflatten(s, Fi):
    g = 128 // Fi
    parts = [s[:, j * Fi:(j + 1) * Fi] for j in range(g)]
    return jnp.stack(parts, axis=1).reshape(s.shape[0] * g, Fi)


def _tc_mid(accf, mp, degp2, bg, Wn, Fi, Fo):
    RB = BLK * Fi // 128

    def body(a0, a1, mb, db, bgb, wb, o):
        d = lax.rsqrt(db[0] + db[1] + 1.0)[:, None]
        s = _unflatten(a0[...] + a1[...], Fi)
        h = jnp.maximum(d * (s + mb[...]) + bgb[...], 0.0)
        o[...] = jnp.dot(h, wb[...], preferred_element_type=jnp.float32) * d

    return pl.pallas_call(
        body,
        grid=(GRID,),
        in_specs=[
            pl.BlockSpec((RB, 128), lambda i: (i, 0)),
            pl.BlockSpec((RB, 128), lambda i: (i + GRID, 0)),
            pl.BlockSpec((BLK, Fi), lambda i: (i, 0)),
            pl.BlockSpec((2, BLK), lambda i: (0, i)),
            pl.BlockSpec((Fi,), lambda i: (0,)),
            pl.BlockSpec((Fi, Fo), lambda i: (0, 0)),
        ],
        out_specs=pl.BlockSpec((BLK, Fo), lambda i: (i, 0)),
        out_shape=jax.ShapeDtypeStruct((NP, Fo), jnp.float32),
    )(accf, accf, mp, degp2, bg, Wn)


def _tc_heads(accf, mp, degp2, bg3, Ws1, bs1, Wv1, bv1, Wv2, bv2):
    RB = BLK * 8 // 128

    def body(a0, a1, mb, db, bgb, ws1, bs1b, wv1, bv1b, wv2, bv2b, a_o, b_o, vsq_o):
        d = lax.rsqrt(db[0] + db[1] + 1.0)[:, None]
        s = _unflatten(a0[...] + a1[...], 8)
        h3 = d * (s + mb[...]) + bgb[...]
        ws1v = ws1[...]
        a_o[...] = jnp.dot(h3, ws1v[:8], preferred_element_type=jnp.float32) + bs1b[...]
        b_o[...] = jnp.dot(h3, ws1v[8:], preferred_element_type=jnp.float32)
        z = jnp.maximum(jnp.dot(h3, wv1[...], preferred_element_type=jnp.float32)
                        + bv1b[...], 0.0)
        t = jnp.dot(z, wv2[...], preferred_element_type=jnp.float32) + bv2b[...]
        vs = 1.0 / (1.0 + jnp.exp(-t))
        vp = 0.9 + 0.2 * vs
        vsq_o[...] = vp * vp

    return pl.pallas_call(
        body,
        grid=(GRID,),
        in_specs=[
            pl.BlockSpec((RB, 128), lambda i: (i, 0)),
            pl.BlockSpec((RB, 128), lambda i: (i + GRID, 0)),
            pl.BlockSpec((BLK, 8), lambda i: (i, 0)),
            pl.BlockSpec((2, BLK), lambda i: (0, i)),
            pl.BlockSpec((8,), lambda i: (0,)),
            pl.BlockSpec((16, 8), lambda i: (0, 0)),
            pl.BlockSpec((8,), lambda i: (0,)),
            pl.BlockSpec((8, 4), lambda i: (0, 0)),
            pl.BlockSpec((4,), lambda i: (0,)),
            pl.BlockSpec((4, 1), lambda i: (0, 0)),
            pl.BlockSpec((1,), lambda i: (0,)),
        ],
        out_specs=[
            pl.BlockSpec((BLK, 8), lambda i: (i, 0)),
            pl.BlockSpec((BLK, 8), lambda i: (i, 0)),
            pl.BlockSpec((BLK, 1), lambda i: (i, 0)),
        ],
        out_shape=[
            jax.ShapeDtypeStruct((NP, 8), jnp.float32),
            jax.ShapeDtypeStruct((NP, 8), jnp.float32),
            jax.ShapeDtypeStruct((NP, 1), jnp.float32),
        ],
    )(accf, accf, mp, degp2, bg3, Ws1, bs1, Wv1, bv1, Wv2, bv2)


def kernel(x, edge_index, W_enc, b_enc, Wg1, bg1, Wg2, bg2, Wg3, bg3,
           Ws1, bs1, Ws2, bs2, Wv1, bv1, Wv2, bv2):
    ei = edge_index.astype(jnp.int32).reshape(2 * E)

    degp = _deg_partials(ei, jnp.zeros((CHUNK,), jnp.float32)).reshape(2, NP)
    mp1 = _tc_encode(x, degp, W_enc, b_enc, Wg1)

    acc1 = _seg_sum_partials(mp1, ei, jnp.zeros((CHUNK, 32), jnp.float32),
                             32, 7).reshape(2 * NP * 32 // 128, 128)
    mp2 = _tc_mid(acc1, mp1, degp, bg1, Wg2, 32, 16)

    acc2 = _seg_sum_partials(mp2, ei, jnp.zeros((CHUNK, 16), jnp.float32),
                             16, 28).reshape(2 * NP * 16 // 128, 128)
    mp3 = _tc_mid(acc2, mp2, degp, bg2, Wg3, 16, 8)

    acc3 = _seg_sum_partials(mp3, ei, jnp.zeros((CHUNK, 8), jnp.float32),
                             8, 28).reshape(2 * NP * 8 // 128, 128)
    A, B, vsq = _tc_heads(acc3, mp3, degp, bg3, Ws1, bs1, Wv1, bv1, Wv2, bv2)

    w2s = jnp.tile(Ws2.reshape(8, 1), (1, 16))
    b2s = jnp.full((16,), 1.0, jnp.float32) * bs2[0]
    sw = _edge_head(A, B, ei, w2s, b2s)

    return sw, vsq[:N, 0]


# speedup vs baseline: 48.0753x; 1.0015x over previous
import functools

import jax
import jax.numpy as jnp
from jax import lax
from jax.experimental import pallas as pl
from jax.experimental.pallas import tpu as pltpu
from jax.experimental.pallas import tpu_sc as plsc

N = 50000
E = 800000
D_IN = 128

NCORES = 2
NSUB = 16
NW = NCORES * NSUB
CHUNK = 128
EPT = E // NW
NFC = EPT // CHUNK
TAIL = EPT - NFC * CHUNK
K = 28
KE = 24
NP = 50176
STRIPE = NP // NSUB

BLK = 6272
GRID = NP // BLK

SFULL = STRIPE // CHUNK
STAIL = STRIPE - SFULL * CHUNK
SCH = SFULL + 1


def _mesh():
    return plsc.VectorSubcoreMesh(
        core_axis_name="c", subcore_axis_name="s",
        num_cores=NCORES, num_subcores=NSUB)


def _zero_stripe(zrow, acc, r0, sem):
    cps = []
    for t in range(SFULL):
        off = pl.multiple_of(r0 + t * CHUNK, 8)
        cps.append(pltpu.async_copy(zrow, acc.at[pl.ds(off, CHUNK)], sem))
    cps.append(pltpu.async_copy(
        zrow.at[pl.ds(0, STAIL)],
        acc.at[pl.ds(r0 + SFULL * CHUNK, STAIL)], sem))
    for cp in cps:
        cp.wait()


def _write_stripe(buf, nslots, acc, out_hbm, r0, obase, sem_l, sem_w):
    chunks = [(t, CHUNK) for t in range(SFULL)] + [(SFULL, STAIL)]
    for w0 in range(0, SCH, nslots):
        wave = chunks[w0:w0 + nslots]
        cps = []
        for slot, (t, ln) in enumerate(wave):
            off = pl.multiple_of(r0 + t * CHUNK, 8)
            cps.append(pltpu.async_copy(acc.at[pl.ds(off, ln)],
                                        buf.at[slot, pl.ds(0, ln)], sem_l))
        for cp in cps:
            cp.wait()
        cps = []
        for slot, (t, ln) in enumerate(wave):
            off = pl.multiple_of(r0 + t * CHUNK, 8)
            cps.append(pltpu.async_copy(buf.at[slot, pl.ds(0, ln)],
                                        out_hbm.at[pl.ds(obase + off, ln)], sem_w))
        for cp in cps:
            cp.wait()


def _deg_partials(ei, zrow):
    @functools.partial(
        pl.kernel,
        out_type=jax.ShapeDtypeStruct((2 * NP,), jnp.float32),
        mesh=_mesh(),
        compiler_params=pltpu.CompilerParams(
            use_tc_tiling_on_sc=False, needs_layout_passes=False),
        scratch_types=[
            pltpu.VMEM((K, CHUNK), jnp.int32),
            pltpu.VMEM((TAIL,), jnp.int32),
            pltpu.VMEM((CHUNK,), jnp.float32),
            pltpu.VMEM((SCH, CHUNK), jnp.float32),
            pltpu.VMEM_SHARED((NP,), jnp.float32),
            pltpu.SemaphoreType.DMA,
            pltpu.SemaphoreType.DMA,
        ],
    )
    def k(ei_hbm, zero_hbm, out_hbm, gdidx, tdidx, ones, buf, acc, sem_i, sem_s):
        c = lax.axis_index("c")
        s = lax.axis_index("s")
        wid = c * NSUB + s
        r0 = pl.multiple_of(s * STRIPE, 8)
        pltpu.sync_copy(zero_hbm, ones)
        _zero_stripe(ones, acc, r0, sem_i)
        plsc.subcore_barrier()
        for i in range(CHUNK // 16):
            ones[pl.ds(16 * i, 16)] = jnp.full((16,), 1.0, jnp.float32)
        ebase = wid * EPT

        def do_group(cbase, nch):
            eb = pl.multiple_of(ebase + cbase * CHUNK, 8)
            cps = [pltpu.async_copy(
                       ei_hbm.at[pl.ds(E + eb + b * CHUNK, CHUNK)],
                       gdidx.at[b], sem_i)
                   for b in range(nch)]
            for cp in cps:
                cp.wait()
            cps = [pltpu.async_copy(ones, acc.at[gdidx.at[b]], sem_s, add=True)
                   for b in range(nch)]
            for cp in cps:
                cp.wait()

        def body(g, carry):
            do_group(g * K, K)
            return carry

        gf = NFC // K
        lax.fori_loop(0, gf, body, 0)
        if NFC % K:
            do_group(gf * K, NFC % K)
        teb = pl.multiple_of(ebase + NFC * CHUNK, 8)
        pltpu.sync_copy(ei_hbm.at[pl.ds(E + teb, TAIL)], tdidx)
        pltpu.sync_copy(ones.at[pl.ds(0, TAIL)], acc.at[tdidx], add=True)
        plsc.subcore_barrier()
        _write_stripe(buf, SCH, acc, out_hbm, r0, c * NP, sem_i, sem_s)

    return k(ei, zrow)


def _seg_sum_partials(mp, ei, zrow, F, kc):
    @functools.partial(
        pl.kernel,
        out_type=jax.ShapeDtypeStruct((2 * NP, F), jnp.float32),
        mesh=_mesh(),
        compiler_params=pltpu.CompilerParams(
            use_tc_tiling_on_sc=False, needs_layout_passes=False),
        scratch_types=[
            pltpu.VMEM((kc * CHUNK,), jnp.int32),
            pltpu.VMEM((kc, CHUNK), jnp.int32),
            pltpu.VMEM((TAIL,), jnp.int32),
            pltpu.VMEM((TAIL,), jnp.int32),
            pltpu.VMEM((kc, CHUNK, F), jnp.float32),
            pltpu.VMEM_SHARED((NP, F), jnp.float32),
            pltpu.SemaphoreType.DMA,
            pltpu.SemaphoreType.DMA,
            pltpu.SemaphoreType.DMA,
        ],
    )
    def k(mp_hbm, ei_hbm, zero_hbm, out_hbm,
          gsidx, gdidx, tsidx, tdidx, rows, acc, sem_i, sem_g, sem_s):
        c = lax.axis_index("c")
        s = lax.axis_index("s")
        wid = c * NSUB + s
        r0 = pl.multiple_of(s * STRIPE, 8)
        pltpu.sync_copy(zero_hbm, rows.at[0])
        _zero_stripe(rows.at[0], acc, r0, sem_i)
        plsc.subcore_barrier()
        ebase = wid * EPT

        def do_group(cbase, nch):
            eb = pl.multiple_of(ebase + cbase * CHUNK, 8)
            cps = [pltpu.async_copy(ei_hbm.at[pl.ds(eb, nch * CHUNK)],
                                    gsidx.at[pl.ds(0, nch * CHUNK)], sem_i)]
            cps += [pltpu.async_copy(
                        ei_hbm.at[pl.ds(E + eb + b * CHUNK, CHUNK)],
                        gdidx.at[b], sem_i)
                    for b in range(nch)]
            for cp in cps:
                cp.wait()
            gcps = [pltpu.async_copy(
                        mp_hbm.at[gsidx.at[pl.ds(b * CHUNK, CHUNK)]],
                        rows.at[b], sem_g)
                    for b in range(nch)]
            scps = []
            for b in range(nch):
                gcps[b].wait()
                scps.append(pltpu.async_copy(rows.at[b], acc.at[gdidx.at[b]],
                                             sem_s, add=True))
            for cp in scps:
                cp.wait()

        def body(g, carry):
            do_group(g * kc, kc)
            return carry

        gf = NFC // kc
        lax.fori_loop(0, gf, body, 0)
        if NFC % kc:
            do_group(gf * kc, NFC % kc)
        teb = pl.multiple_of(ebase + NFC * CHUNK, 8)
        pltpu.sync_copy(ei_hbm.at[pl.ds(teb, TAIL)], tsidx)
        pltpu.sync_copy(ei_hbm.at[pl.ds(E + teb, TAIL)], tdidx)
        pltpu.async_copy(mp_hbm.at[tsidx], rows.at[0, pl.ds(0, TAIL)], sem_g).wait()
        pltpu.sync_copy(rows.at[0, pl.ds(0, TAIL)], acc.at[tdidx], add=True)
        plsc.subcore_barrier()
        _write_stripe(rows, kc, acc, out_hbm, r0, c * NP, sem_g, sem_s)

    return k(mp, ei, zrow)


def _seg_sum_piped(mp, ei, zrow, F, kc):
    @functools.partial(
        pl.kernel,
        out_type=jax.ShapeDtypeStruct((2 * NP, F), jnp.float32),
        mesh=_mesh(),
        compiler_params=pltpu.CompilerParams(
            use_tc_tiling_on_sc=False, needs_layout_passes=False),
        scratch_types=[
            pltpu.VMEM((4, kc * CHUNK), jnp.int32),
            pltpu.VMEM((4, kc, CHUNK), jnp.int32),
            pltpu.VMEM((TAIL,), jnp.int32),
            pltpu.VMEM((TAIL,), jnp.int32),
            pltpu.VMEM((2, kc, CHUNK, F), jnp.float32),
            pltpu.VMEM_SHARED((NP, F), jnp.float32),
            pltpu.SemaphoreType.DMA,
            pltpu.SemaphoreType.DMA,
            pltpu.SemaphoreType.DMA,
        ],
    )
    def k(mp_hbm, ei_hbm, zero_hbm, out_hbm,
          gsidx, gdidx, tsidx, tdidx, rows, acc, sem_i, sem_g, sem_s):
        c = lax.axis_index("c")
        s = lax.axis_index("s")
        wid = c * NSUB + s
        r0 = pl.multiple_of(s * STRIPE, 8)
        pltpu.sync_copy(zero_hbm, rows.at[0, 0])
        _zero_stripe(rows.at[0, 0], acc, r0, sem_i)
        plsc.subcore_barrier()
        ebase = wid * EPT

        grps = [(g * kc, kc) for g in range(NFC // kc)]
        if NFC % kc:
            grps.append(((NFC // kc) * kc, NFC % kc))
        ng = len(grps)

        def issue_idx(gi):
            cb, nch = grps[gi]
            qb = gi % 4
            eb = pl.multiple_of(ebase + cb * CHUNK, 8)
            cps = [pltpu.async_copy(ei_hbm.at[pl.ds(eb, nch * CHUNK)],
                                    gsidx.at[qb, pl.ds(0, nch * CHUNK)], sem_i)]
            cps += [pltpu.async_copy(
                        ei_hbm.at[pl.ds(E + eb + b * CHUNK, CHUNK)],
                        gdidx.at[qb, b], sem_i)
                    for b in range(nch)]
            return cps

        icps = {0: issue_idx(0)}
        if ng > 1:
            icps[1] = issue_idx(1)
        pend = {}
        for gi in range(ng):
            qb, pb, nch = gi % 4, gi % 2, grps[gi][1]
            for cp in icps.pop(gi):
                cp.wait()
            if pb in pend:
                for cp in pend.pop(pb):
                    cp.wait()
            gcps = [pltpu.async_copy(
                        mp_hbm.at[gsidx.at[qb, pl.ds(b * CHUNK, CHUNK)]],
                        rows.at[pb, b], sem_g)
                    for b in range(nch)]
            if gi + 2 < ng:
                icps[gi + 2] = issue_idx(gi + 2)
            scps = []
            for b in range(nch):
                gcps[b].wait()
                scps.append(pltpu.async_copy(rows.at[pb, b],
                                             acc.at[gdidx.at[qb, b]],
                                             sem_s, add=True))
            pend[pb] = scps
        for cps in pend.values():
            for cp in cps:
                cp.wait()
        teb = pl.multiple_of(ebase + NFC * CHUNK, 8)
        pltpu.sync_copy(ei_hbm.at[pl.ds(teb, TAIL)], tsidx)
        pltpu.sync_copy(ei_hbm.at[pl.ds(E + teb, TAIL)], tdidx)
        pltpu.async_copy(mp_hbm.at[tsidx], rows.at[0, 0, pl.ds(0, TAIL)], sem_g).wait()
        pltpu.sync_copy(rows.at[0, 0, pl.ds(0, TAIL)], acc.at[tdidx], add=True)
        plsc.subcore_barrier()
        _write_stripe(rows.at[0], kc, acc, out_hbm, r0, c * NP, sem_g, sem_s)

    return k(mp, ei, zrow)


def _edge_head(A, B, ei, w2s, b2s):
    @functools.partial(
        pl.kernel,
        out_type=jax.ShapeDtypeStruct((E,), jnp.float32),
        mesh=_mesh(),
        compiler_params=pltpu.CompilerParams(
            use_tc_tiling_on_sc=False, needs_layout_passes=False),
        scratch_types=[
            pltpu.VMEM((3, KE * CHUNK), jnp.int32),
            pltpu.VMEM((3, KE * CHUNK), jnp.int32),
            pltpu.VMEM((TAIL,), jnp.int32),
            pltpu.VMEM((TAIL,), jnp.int32),
            pltpu.VMEM((2, KE, CHUNK, 8), jnp.float32),
            pltpu.VMEM((2, KE, CHUNK, 8), jnp.float32),
            pltpu.VMEM((8, 16), jnp.float32),
            pltpu.VMEM((16,), jnp.float32),
            pltpu.VMEM((2, KE * CHUNK), jnp.float32),
            pltpu.SemaphoreType.DMA,
            pltpu.SemaphoreType.DMA,
            pltpu.SemaphoreType.DMA,
        ],
    )
    def k(a_hbm, b_hbm, ei_hbm, w_hbm, b2_hbm, out_hbm,
          gsidx, gdidx, tsidx, tdidx, ra, rb, wv, b2v, ob, sem_i, sem_g, sem_w):
        c = lax.axis_index("c")
        s = lax.axis_index("s")
        wid = c * NSUB + s
        pltpu.sync_copy(w_hbm, wv)
        pltpu.sync_copy(b2_hbm, b2v)
        wsp = [wv[j, :] for j in range(8)]
        b2 = b2v[...]
        iota = lax.iota(jnp.int32, 16)
        ebase = wid * EPT

        def compute_chunk(pb, b, nvec):
            bb = jnp.full((16,), b, jnp.int32)
            for gg in range(nvec):
                ridx = iota + (gg * 16)
                acc = jnp.zeros((16,), jnp.float32)
                for j in range(8):
                    cj = jnp.full((16,), j, jnp.int32)
                    av = plsc.load_gather(ra.at[pb], [bb, ridx, cj])
                    bv = plsc.load_gather(rb.at[pb], [bb, ridx, cj])
                    acc = acc + jnp.maximum(av + bv, 0.0) * wsp[j]
                t = acc + b2
                ob[pb, pl.ds(b * CHUNK + gg * 16, 16)] = 1.0 / (1.0 + jnp.exp(-t))

        grps = [(g * KE, KE) for g in range(NFC // KE)]
        if NFC % KE:
            grps.append(((NFC // KE) * KE, NFC % KE))
        ng = len(grps)

        def issue_idx(gi):
            cb, nch = grps[gi]
            qb = gi % 3
            eb = pl.multiple_of(ebase + cb * CHUNK, 8)
            nw = nch * CHUNK
            return (pltpu.async_copy(ei_hbm.at[pl.ds(eb, nw)],
                                     gsidx.at[qb, pl.ds(0, nw)], sem_i),
                    pltpu.async_copy(ei_hbm.at[pl.ds(E + eb, nw)],
                                     gdidx.at[qb, pl.ds(0, nw)], sem_i))

        def issue_gathers(gi):
            qb, pb, nch = gi % 3, gi % 2, grps[gi][1]
            for b in range(nch):
                sl = pl.ds(b * CHUNK, CHUNK)
                pltpu.async_copy(a_hbm.at[gsidx.at[qb, sl]], ra.at[pb, b], sem_g)
                pltpu.async_copy(b_hbm.at[gdidx.at[qb, sl]], rb.at[pb, b], sem_g)

        def compute_group(gi):
            pb, (cb, nch) = gi % 2, grps[gi]

            def chunk(b, carry2):
                pltpu.make_async_copy(a_hbm.at[pl.ds(0, CHUNK)],
                                      ra.at[0, 0], sem_g).wait()
                pltpu.make_async_copy(b_hbm.at[pl.ds(0, CHUNK)],
                                      rb.at[0, 0], sem_g).wait()
                compute_chunk(pb, b, CHUNK // 16)
                return carry2

            lax.fori_loop(0, nch, chunk, 0)
            eb = pl.multiple_of(ebase + cb * CHUNK, 8)
            nw = nch * CHUNK
            return pltpu.async_copy(ob.at[pb, pl.ds(0, nw)],
                                    out_hbm.at[pl.ds(eb, nw)], sem_w)

        icps = {0: issue_idx(0)}
        if ng > 1:
            icps[1] = issue_idx(1)
        stores = {}
        for gi in range(ng):
            icps[gi][0].wait()
            icps[gi][1].wait()
            issue_gathers(gi)
            if gi + 2 < ng:
                icps[gi + 2] = issue_idx(gi + 2)
            if gi % 2 in stores:
                stores[gi % 2].wait()
            stores[gi % 2] = compute_group(gi)
        teb = pl.multiple_of(ebase + NFC * CHUNK, 8)
        pltpu.sync_copy(ei_hbm.at[pl.ds(teb, TAIL)], tsidx)
        pltpu.sync_copy(ei_hbm.at[pl.ds(E + teb, TAIL)], tdidx)
        if 0 in stores:
            stores[0].wait()
        cpa = pltpu.async_copy(a_hbm.at[tsidx], ra.at[0, 0, pl.ds(0, TAIL)], sem_g)
        cpb = pltpu.async_copy(b_hbm.at[tdidx], rb.at[0, 0, pl.ds(0, TAIL)], sem_g)
        cpa.wait()
        cpb.wait()
        compute_chunk(0, 0, (TAIL + 15) // 16)
        pltpu.sync_copy(ob.at[0, pl.ds(0, TAIL)], out_hbm.at[pl.ds(teb, TAIL)])
        if 1 in stores:
            stores[1].wait()

    return k(A, B, ei, w2s, b2s)


def _tc_encode(x, degp2, W_enc, b_enc, Wg1):
    def body(xb, degb, we, be, wg, mp_o):
        deg = degb[0] + degb[1] + 1.0
        dis = lax.rsqrt(deg)
        h0 = jnp.maximum(jnp.dot(xb[...], we[...],
                                 preferred_element_type=jnp.float32) + be[...], 0.0)
        mp_o[...] = jnp.dot(h0, wg[...], preferred_element_type=jnp.float32) * dis[:, None]

    return pl.pallas_call(
        body,
        grid=(GRID,),
        in_specs=[
            pl.BlockSpec((BLK, D_IN), lambda i: (i, 0)),
            pl.BlockSpec((2, BLK), lambda i: (0, i)),
            pl.BlockSpec((D_IN, 64), lambda i: (0, 0)),
            pl.BlockSpec((64,), lambda i: (0,)),
            pl.BlockSpec((64, 32), lambda i: (0, 0)),
        ],
        out_specs=pl.BlockSpec((BLK, 32), lambda i: (i, 0)),
        out_shape=jax.ShapeDtypeStruct((NP, 32), jnp.float32),
    )(x, degp2, W_enc, b_enc, Wg1)


def _unflatten(s, Fi):
    g = 128 // Fi
    parts = [s[:, j * Fi:(j + 1) * Fi] for j in range(g)]
    return jnp.stack(parts, axis=1).reshape(s.shape[0] * g, Fi)


def _tc_mid(accf, mp, degp2, bg, Wn, Fi, Fo):
    RB = BLK * Fi // 128

    def body(a0, a1, mb, db, bgb, wb, o):
        d = lax.rsqrt(db[0] + db[1] + 1.0)[:, None]
        s = _unflatten(a0[...] + a1[...], Fi)
        h = jnp.maximum(d * (s + mb[...]) + bgb[...], 0.0)
        o[...] = jnp.dot(h, wb[...], preferred_element_type=jnp.float32) * d

    return pl.pallas_call(
        body,
        grid=(GRID,),
        in_specs=[
            pl.BlockSpec((RB, 128), lambda i: (i, 0)),
            pl.BlockSpec((RB, 128), lambda i: (i + GRID, 0)),
            pl.BlockSpec((BLK, Fi), lambda i: (i, 0)),
            pl.BlockSpec((2, BLK), lambda i: (0, i)),
            pl.BlockSpec((Fi,), lambda i: (0,)),
            pl.BlockSpec((Fi, Fo), lambda i: (0, 0)),
        ],
        out_specs=pl.BlockSpec((BLK, Fo), lambda i: (i, 0)),
        out_shape=jax.ShapeDtypeStruct((NP, Fo), jnp.float32),
    )(accf, accf, mp, degp2, bg, Wn)


def _tc_heads(accf, mp, degp2, bg3, Ws1, bs1, Wv1, bv1, Wv2, bv2):
    RB = BLK * 8 // 128

    def body(a0, a1, mb, db, bgb, ws1, bs1b, wv1, bv1b, wv2, bv2b, a_o, b_o, vsq_o):
        d = lax.rsqrt(db[0] + db[1] + 1.0)[:, None]
        s = _unflatten(a0[...] + a1[...], 8)
        h3 = d * (s + mb[...]) + bgb[...]
        ws1v = ws1[...]
        a_o[...] = jnp.dot(h3, ws1v[:8], preferred_element_type=jnp.float32) + bs1b[...]
        b_o[...] = jnp.dot(h3, ws1v[8:], preferred_element_type=jnp.float32)
        z = jnp.maximum(jnp.dot(h3, wv1[...], preferred_element_type=jnp.float32)
                        + bv1b[...], 0.0)
        t = jnp.dot(z, wv2[...], preferred_element_type=jnp.float32) + bv2b[...]
        vs = 1.0 / (1.0 + jnp.exp(-t))
        vp = 0.9 + 0.2 * vs
        vsq_o[...] = vp * vp

    return pl.pallas_call(
        body,
        grid=(GRID,),
        in_specs=[
            pl.BlockSpec((RB, 128), lambda i: (i, 0)),
            pl.BlockSpec((RB, 128), lambda i: (i + GRID, 0)),
            pl.BlockSpec((BLK, 8), lambda i: (i, 0)),
            pl.BlockSpec((2, BLK), lambda i: (0, i)),
            pl.BlockSpec((8,), lambda i: (0,)),
            pl.BlockSpec((16, 8), lambda i: (0, 0)),
            pl.BlockSpec((8,), lambda i: (0,)),
            pl.BlockSpec((8, 4), lambda i: (0, 0)),
            pl.BlockSpec((4,), lambda i: (0,)),
            pl.BlockSpec((4, 1), lambda i: (0, 0)),
            pl.BlockSpec((1,), lambda i: (0,)),
        ],
        out_specs=[
            pl.BlockSpec((BLK, 8), lambda i: (i, 0)),
            pl.BlockSpec((BLK, 8), lambda i: (i, 0)),
            pl.BlockSpec((BLK, 1), lambda i: (i, 0)),
        ],
        out_shape=[
            jax.ShapeDtypeStruct((NP, 8), jnp.float32),
            jax.ShapeDtypeStruct((NP, 8), jnp.float32),
            jax.ShapeDtypeStruct((NP, 1), jnp.float32),
        ],
    )(accf, accf, mp, degp2, bg3, Ws1, bs1, Wv1, bv1, Wv2, bv2)


def kernel(x, edge_index, W_enc, b_enc, Wg1, bg1, Wg2, bg2, Wg3, bg3,
           Ws1, bs1, Ws2, bs2, Wv1, bv1, Wv2, bv2):
    ei = edge_index.astype(jnp.int32).reshape(2 * E)

    degp = _deg_partials(ei, jnp.zeros((CHUNK,), jnp.float32)).reshape(2, NP)
    mp1 = _tc_encode(x, degp, W_enc, b_enc, Wg1)

    acc1 = _seg_sum_partials(mp1, ei, jnp.zeros((CHUNK, 32), jnp.float32),
                             32, 7).reshape(2 * NP * 32 // 128, 128)
    mp2 = _tc_mid(acc1, mp1, degp, bg1, Wg2, 32, 16)

    acc2 = _seg_sum_piped(mp2, ei, jnp.zeros((CHUNK, 16), jnp.float32),
                          16, 12).reshape(2 * NP * 16 // 128, 128)
    mp3 = _tc_mid(acc2, mp2, degp, bg2, Wg3, 16, 8)

    acc3 = _seg_sum_piped(mp3, ei, jnp.zeros((CHUNK, 8), jnp.float32),
                          8, 24).reshape(2 * NP * 8 // 128, 128)
    A, B, vsq = _tc_heads(acc3, mp3, degp, bg3, Ws1, bs1, Wv1, bv1, Wv2, bv2)

    w2s = jnp.tile(Ws2.reshape(8, 1), (1, 16))
    b2s = jnp.full((16,), 1.0, jnp.float32) * bs2[0]
    sw = _edge_head(A, B, ei, w2s, b2s)

    return sw, vsq[:N, 0]
